# Initial kernel scaffold; baseline (speedup 1.0000x reference)
import jax
import jax.numpy as jnp
from jax.experimental import pallas as pl


def kernel(x1, edge_index1, e1, u1, batch1, x2, edge_index2, e2, u2, batch2, params):
    raise NotImplementedError("write your pallas kernel here")


# speedup vs baseline: 1.1425x; 1.1425x over previous
import functools

import jax
import jax.numpy as jnp
from jax import lax
from jax.experimental import pallas as pl
from jax.experimental.pallas import tpu as pltpu
from jax.experimental.pallas import tpu_sc as plsc

F32 = jnp.float32
I32 = jnp.int32
SDS = jax.ShapeDtypeStruct
PREC = lax.Precision.HIGHEST

N1 = 10000
E1 = 160000
NG = 16
H = 128

BLK = 1000


def _dot(a, b):
    return lax.dot_general(a, b, (((a.ndim - 1,), (0,)), ((), ())),
                           precision=PREC, preferred_element_type=F32)


def _mlp23(h, l2, l3):
    (w2, b2), (w3, b3) = l2, l3
    h = jnp.maximum(_dot(h, w2[...]) + b2[...], 0.0)
    return _dot(h, w3[...]) + b3[...]


def _wspec(arr):
    nd = arr.ndim
    return pl.BlockSpec(arr.shape, lambda i: (0,) * nd)


def _full_w(params):
    return [(w, b.reshape(1, -1)) for (w, b) in params]


def _enc(x, layers):
    R = x.shape[0]
    blk = BLK if R % BLK == 0 else R
    (w1, b1), l2, l3 = layers

    def body(x_ref, w1r, b1r, w2r, b2r, w3r, b3r, o_ref):
        h = jnp.maximum(_dot(x_ref[...], w1r[...]) + b1r[...], 0.0)
        o_ref[...] = _mlp23(h, (w2r, b2r), (w3r, b3r))

    ins = [x, w1, b1, l2[0], l2[1], l3[0], l3[1]]
    specs = [pl.BlockSpec((blk, x.shape[1]), lambda i: (i, 0))] + [_wspec(a) for a in ins[1:]]
    return pl.pallas_call(
        body, grid=(R // blk,), in_specs=specs,
        out_specs=pl.BlockSpec((blk, H), lambda i: (i, 0)),
        out_shape=SDS((R, H), F32))(*ins)


def _a_proj(x_parts, w1x):
    R = x_parts[0].shape[0]
    np_ = len(x_parts)

    def body(*refs):
        xr = refs[:np_]
        wr = refs[np_]
        o_ref = refs[np_ + 1]
        xc = jnp.concatenate([r[...] for r in xr], axis=1)
        o_ref[...] = _dot(xc, wr[...])

    ins = list(x_parts) + [w1x]
    specs = [pl.BlockSpec((BLK, H), lambda i: (i, 0)) for _ in x_parts] + [_wspec(w1x)]
    return pl.pallas_call(
        body, grid=(R // BLK,), in_specs=specs,
        out_specs=pl.BlockSpec((BLK, H), lambda i: (i, 0)),
        out_shape=SDS((R, H), F32))(*ins)


def _graph_starts(b_a, b_b):
    nblk = b_a.shape[0]

    def body(bar, bbr, row_ref, col_ref):
        i = pl.program_id(0)
        ids_a = bar[...][0]
        ids_b = bbr[...][0]
        lt = (ids_a < lax.broadcasted_iota(I32, (BLK, NG), 1)).astype(F32)
        ltT = (lax.broadcasted_iota(I32, (NG, BLK), 0) > ids_b).astype(F32)

        @pl.when(i == 0)
        def _():
            row_ref[...] = jnp.zeros((1, NG), F32)
            col_ref[...] = jnp.zeros((NG, 1), F32)

        row_ref[...] += _dot(jnp.ones((1, BLK), F32), lt)
        col_ref[...] += _dot(ltT, jnp.ones((BLK, 1), F32))

    return pl.pallas_call(
        body, grid=(nblk,),
        in_specs=[pl.BlockSpec((1, BLK, 1), lambda i: (i, 0, 0)),
                  pl.BlockSpec((1, 1, BLK), lambda i: (i, 0, 0))],
        out_specs=(pl.BlockSpec((1, NG), lambda i: (0, 0)),
                   pl.BlockSpec((NG, 1), lambda i: (0, 0))),
        out_shape=(SDS((1, NG), F32), SDS((NG, 1), F32)))(b_a, b_b)


def _seg_onehots(ids_a, ids_b, cum_row, cum_col):
    fa = ids_a.astype(F32)
    fb = ids_b.astype(F32)
    ge = (fa >= cum_row).astype(F32)
    geT = (fb >= cum_col).astype(F32)
    oh = ge - jnp.concatenate([ge[:, 1:], jnp.zeros((ge.shape[0], 1), F32)], 1)
    ohT = geT - jnp.concatenate([geT[1:, :], jnp.zeros((1, geT.shape[1]), F32)], 0)
    return oh, ohT


def _edge(ad, as_, e_parts, src_a, src_b, cum, u_parts, w1e, w1u, b1, l2, l3,
          ew_mult=None):
    E = ad.shape[0]
    cum_row, cum_col = cum
    ne, nu = len(e_parts), len(u_parts)
    with_ew = ew_mult is not None

    def body(*refs):
        it = iter(refs)
        adr = next(it); asr = next(it)
        ers = [next(it) for _ in range(ne)]
        sar = next(it); sbr = next(it)
        crr = next(it); ccr = next(it)
        urs = [next(it) for _ in range(nu)]
        w1er = next(it); w1ur = next(it); b1r = next(it)
        w2r = next(it); b2r = next(it); w3r = next(it); b3r = next(it)
        mr = next(it) if with_ew else None
        e2_ref = next(it)
        aggg_ref = next(it)
        aggew_ref = next(it) if with_ew else None

        i = pl.program_id(0)
        ecat = jnp.concatenate([r[...] for r in ers], axis=1)
        ucat = jnp.concatenate([r[...] for r in urs], axis=1)
        gu = _dot(ucat, w1ur[...]) + b1r[...]
        oh, ohT = _seg_onehots(sar[...][0], sbr[...][0], crr[...], ccr[...])
        h = adr[...] - asr[...] + _dot(ecat, w1er[...]) + _dot(oh, gu)
        h = jnp.maximum(h, 0.0)
        e2 = _mlp23(h, (w2r, b2r), (w3r, b3r))
        e2_ref[...] = e2

        @pl.when(i == 0)
        def _():
            aggg_ref[...] = jnp.zeros((NG, H), F32)
            if with_ew:
                aggew_ref[...] = jnp.zeros((NG, H), F32)

        aggg_ref[...] += _dot(ohT, e2)
        if with_ew:
            aggew_ref[...] += _dot(ohT, e2 * mr[...])

    ins = [ad, as_] + list(e_parts) + [src_a, src_b, cum_row, cum_col] + \
        list(u_parts) + [w1e, w1u, b1, l2[0], l2[1], l3[0], l3[1]]
    specs = [pl.BlockSpec((BLK, H), lambda i: (i, 0)) for _ in range(2 + ne)]
    specs += [pl.BlockSpec((1, BLK, 1), lambda i: (i, 0, 0)),
              pl.BlockSpec((1, 1, BLK), lambda i: (i, 0, 0))]
    specs += [_wspec(a) for a in ins[4 + ne:]]
    if with_ew:
        ins.append(ew_mult)
        specs.append(pl.BlockSpec((BLK, H), lambda i: (i, 0)))

    out_shape = [SDS((E, H), F32), SDS((NG, H), F32)]
    out_specs = [pl.BlockSpec((BLK, H), lambda i: (i, 0)),
                 pl.BlockSpec((NG, H), lambda i: (0, 0))]
    if with_ew:
        out_shape.append(SDS((NG, H), F32))
        out_specs.append(pl.BlockSpec((NG, H), lambda i: (0, 0)))
    return pl.pallas_call(
        body, grid=(E // BLK,), in_specs=specs, out_specs=tuple(out_specs),
        out_shape=tuple(out_shape))(*ins)


def _node(x_parts, aggn, b_a, b_b, u_parts, w1xa, w1u, b1, l2, l3, xw_mult=None):
    R = aggn.shape[0]
    nx, nu = len(x_parts), len(u_parts)
    with_xw = xw_mult is not None

    def body(*refs):
        it = iter(refs)
        xrs = [next(it) for _ in range(nx)]
        aggr = next(it)
        bar = next(it); bbr = next(it)
        urs = [next(it) for _ in range(nu)]
        w1r = next(it); w1ur = next(it); b1r = next(it)
        w2r = next(it); b2r = next(it); w3r = next(it); b3r = next(it)
        mr = next(it) if with_xw else None
        x2_ref = next(it)
        aggx_ref = next(it)
        aggxw_ref = next(it) if with_xw else None

        i = pl.program_id(0)
        xc = jnp.concatenate([r[...] for r in xrs] + [aggr[...]], axis=1)
        ucat = jnp.concatenate([r[...] for r in urs], axis=1)
        gu = _dot(ucat, w1ur[...]) + b1r[...]
        ids_a = bar[...][0]
        ids_b = bbr[...][0]
        oh = (ids_a == lax.broadcasted_iota(I32, (BLK, NG), 1)).astype(F32)
        ohT = (ids_b == lax.broadcasted_iota(I32, (NG, BLK), 0)).astype(F32)
        h = jnp.maximum(_dot(xc, w1r[...]) + _dot(oh, gu), 0.0)
        x2 = _mlp23(h, (w2r, b2r), (w3r, b3r))
        x2_ref[...] = x2

        @pl.when(i == 0)
        def _():
            aggx_ref[...] = jnp.zeros((NG, H), F32)
            if with_xw:
                aggxw_ref[...] = jnp.zeros((NG, H), F32)

        aggx_ref[...] += _dot(ohT, x2)
        if with_xw:
            aggxw_ref[...] += _dot(ohT, x2 * mr[...])

    ins = list(x_parts) + [aggn, b_a, b_b] + list(u_parts) + \
        [w1xa, w1u, b1, l2[0], l2[1], l3[0], l3[1]]
    specs = [pl.BlockSpec((BLK, H), lambda i: (i, 0)) for _ in range(nx + 1)]
    specs += [pl.BlockSpec((1, BLK, 1), lambda i: (i, 0, 0)),
              pl.BlockSpec((1, 1, BLK), lambda i: (i, 0, 0))]
    specs += [_wspec(a) for a in ins[nx + 3:]]
    if with_xw:
        ins.append(xw_mult)
        specs.append(pl.BlockSpec((BLK, H), lambda i: (i, 0)))

    out_shape = [SDS((R, H), F32), SDS((NG, H), F32)]
    out_specs = [pl.BlockSpec((BLK, H), lambda i: (i, 0)),
                 pl.BlockSpec((NG, H), lambda i: (0, 0))]
    if with_xw:
        out_shape.append(SDS((NG, H), F32))
        out_specs.append(pl.BlockSpec((NG, H), lambda i: (0, 0)))
    return pl.pallas_call(
        body, grid=(R // BLK,), in_specs=specs, out_specs=tuple(out_specs),
        out_shape=tuple(out_shape))(*ins)


def _useg(aggx, aggg, u_parts, layers):
    nu = len(u_parts)
    (w1, b1), l2, l3 = layers

    def body(*refs):
        it = iter(refs)
        ax = next(it); ag = next(it)
        urs = [next(it) for _ in range(nu)]
        w1r = next(it); b1r = next(it)
        w2r = next(it); b2r = next(it); w3r = next(it); b3r = next(it)
        o_ref = next(it)
        xin = jnp.concatenate([ax[...], ag[...]] + [r[...] for r in urs], axis=1)
        h = jnp.maximum(_dot(xin, w1r[...]) + b1r[...], 0.0)
        o_ref[...] = _mlp23(h, (w2r, b2r), (w3r, b3r))

    ins = [aggx, aggg] + list(u_parts) + [w1, b1, l2[0], l2[1], l3[0], l3[1]]
    specs = [_wspec(a) for a in ins]
    return pl.pallas_call(
        body, grid=(1,), in_specs=specs,
        out_specs=pl.BlockSpec((NG, H), lambda i: (0, 0)),
        out_shape=SDS((NG, H), F32))(*ins)


def _final(aggxw, aggew, u_a, u_h, lay_agg, lay_fin):
    def body(axw, aew, uar, uhr,
             aw1, ab1, aw2, ab2, aw3, ab3,
             fw1, fb1, fw2, fb2, fw3, fb3, o_ref):
        uw = uar[...] * uhr[...]
        gin = jnp.concatenate([axw[...], aew[...], uw], axis=1)
        h = jnp.maximum(_dot(gin, aw1[...]) + ab1[...], 0.0)
        g = _mlp23(h, (aw2, ab2), (aw3, ab3))
        d = g[0:8, :] - g[8:16, :]
        h2 = jnp.maximum(_dot(d, fw1[...]) + fb1[...], 0.0)
        o_ref[...] = _mlp23(h2, (fw2, fb2), (fw3, fb3))

    (aw1, ab1), (aw2, ab2), (aw3, ab3) = lay_agg
    (fw1, fb1), (fw2, fb2), (fw3, fb3) = lay_fin
    ins = [aggxw, aggew, u_a, u_h,
           aw1, ab1, aw2, ab2, aw3, ab3, fw1, fb1, fw2, fb2, fw3, fb3]
    specs = [_wspec(a) for a in ins]
    return pl.pallas_call(
        body, grid=(1,), in_specs=specs,
        out_specs=pl.BlockSpec((8, 64), lambda i: (0, 0)),
        out_shape=SDS((8, 64), F32))(*ins)


_MESH = dict(core_axis_name="c", subcore_axis_name="s")
_NC, _NS = 2, 16
_NW = _NC * _NS


def _sc_gather2(a, dst, src):
    E = dst.shape[0]
    CH = 128
    nch = E // CH
    iters = -(-nch // _NW)

    @functools.partial(
        pl.kernel,
        out_type=(SDS((E, H), F32), SDS((E, H), F32)),
        mesh=plsc.VectorSubcoreMesh(**_MESH),
        scratch_types=[pltpu.VMEM((CH,), I32), pltpu.VMEM((CH,), I32),
                       pltpu.VMEM((CH, H), F32), pltpu.VMEM((CH, H), F32),
                       pltpu.SemaphoreType.DMA, pltpu.SemaphoreType.DMA],
    )
    def k(a_hbm, dst_hbm, src_hbm, ad_hbm, as_hbm, i1, i2, r1, r2, s1, s2):
        cid = lax.axis_index("c")
        sid = lax.axis_index("s")
        wid = sid * _NC + cid

        def body(t, carry):
            ch = wid + t * _NW

            @pl.when(ch < nch)
            def _():
                base = ch * CH
                pltpu.sync_copy(dst_hbm.at[pl.ds(base, CH)], i1)
                pltpu.sync_copy(src_hbm.at[pl.ds(base, CH)], i2)
                c1 = pltpu.async_copy(a_hbm.at[i1], r1, s1)
                c2 = pltpu.async_copy(a_hbm.at[i2], r2, s2)
                c1.wait()
                c2.wait()
                pltpu.sync_copy(r1, ad_hbm.at[pl.ds(base, CH)])
                pltpu.sync_copy(r2, as_hbm.at[pl.ds(base, CH)])

            return carry

        lax.fori_loop(0, iters, body, 0)

    return k(a, dst, src)


def _sc_scatter(e2, dst):
    E = dst.shape[0]
    CH = 128
    epc = E // _NC
    nch = epc // CH
    iters = -(-nch // _NS)
    HALF = N1 // 2
    DUMMY = HALF
    rsmall, rbig = 312, 320

    @functools.partial(
        pl.kernel,
        out_type=SDS((2 * N1, H), F32),
        mesh=plsc.VectorSubcoreMesh(**_MESH),
        scratch_types=[pltpu.VMEM((CH,), I32),
                       pltpu.VMEM((CH, H), F32),
                       pltpu.VMEM((rbig, H), F32),
                       pltpu.VMEM_SHARED((HALF + 120, H), F32)],
    )
    def k(e2_hbm, dst_hbm, aggn_hbm, ibuf, rbuf, zbuf, acc):
        cid = lax.axis_index("c")
        sid = lax.axis_index("s")

        def zrow(r, carry):
            for j in range(H // 16):
                zbuf[r, pl.ds(j * 16, 16)] = jnp.zeros((16,), F32)
            return carry

        lax.fori_loop(0, rbig, zrow, 0)
        rbase = sid * rsmall
        last = _NS - 1

        for h in range(2):
            lo = cid * N1 + h * HALF

            @pl.when(sid < last)
            def _():
                pltpu.sync_copy(zbuf.at[pl.ds(0, rsmall)],
                                acc.at[pl.ds(rbase, rsmall)])

            @pl.when(sid == last)
            def _():
                pltpu.sync_copy(zbuf, acc.at[pl.ds(last * rsmall, rbig)])

            plsc.subcore_barrier()

            def body(t, carry):
                ch = sid + t * _NS

                @pl.when(ch < nch)
                def _():
                    base = cid * epc + ch * CH
                    pltpu.sync_copy(e2_hbm.at[pl.ds(base, CH)], rbuf)
                    pltpu.sync_copy(dst_hbm.at[pl.ds(base, CH)], ibuf)
                    for j in range(CH // 16):
                        v = ibuf[pl.ds(j * 16, 16)] - lo
                        ok = (v >= 0) & (v < HALF)
                        ibuf[pl.ds(j * 16, 16)] = jnp.where(ok, v, DUMMY)
                    pltpu.sync_copy(rbuf, acc.at[ibuf], add=True)

                return carry

            lax.fori_loop(0, iters, body, 0)
            plsc.subcore_barrier()

            @pl.when(sid < last)
            def _():
                pltpu.sync_copy(acc.at[pl.ds(rbase, rsmall)],
                                aggn_hbm.at[pl.ds(lo + rbase, rsmall)])

            @pl.when(sid == last)
            def _():
                pltpu.sync_copy(acc.at[pl.ds(last * rsmall, rbig)],
                                aggn_hbm.at[pl.ds(lo + last * rsmall, rbig)])

            plsc.subcore_barrier()

    return k(e2, dst)


def _meta(pe, px, pu, x_parts, e_parts, u_parts, gidx, attention=False):
    dst, src, src_a, src_b, cum, b_a, b_b = gidx
    npart = len(x_parts)
    K = npart * H

    (we1, be1), el2, el3 = _full_w(pe)
    (wx1, bx1), xl2, xl3 = _full_w(px)

    w1x = we1[0:K]
    w1e = we1[K:2 * K]
    w1u = we1[2 * K:3 * K]
    a = _a_proj(x_parts, w1x)
    ad, as_ = _sc_gather2(a, dst, src)

    ew_mult = e_parts[0] if attention else None
    eouts = _edge(ad, as_, e_parts, src_a, src_b, cum, u_parts,
                  w1e, w1u, be1, el2, el3, ew_mult=ew_mult)
    if attention:
        e2, aggg, aggew = eouts
    else:
        e2, aggg = eouts
        aggew = None

    aggn = _sc_scatter(e2, dst)

    w1xa = wx1[0:K + H]
    w1xu = wx1[K + H:K + H + npart * H]
    xw_mult = x_parts[0] if attention else None
    nouts = _node(x_parts, aggn, b_a, b_b, u_parts,
                  w1xa, w1xu, bx1, xl2, xl3, xw_mult=xw_mult)
    if attention:
        x2, aggx, aggxw = nouts
    else:
        x2, aggx = nouts
        aggxw = None

    u2 = _useg(aggx, aggg, u_parts, _full_w(pu))
    return x2, e2, u2, (aggxw, aggew)


def kernel(x1, edge_index1, e1, u1, batch1, x2, edge_index2, e2, u2, batch2, params):
    x = jnp.concatenate([x1, x2], axis=0)
    e = jnp.concatenate([e1, e2], axis=0)
    u = jnp.concatenate([u1, u2], axis=0)
    batch = jnp.concatenate(
        [batch1.astype(I32), batch2.astype(I32) + 8], axis=0)
    src = jnp.concatenate(
        [edge_index1[0].astype(I32), edge_index2[0].astype(I32) + N1])
    dst = jnp.concatenate(
        [edge_index1[1].astype(I32), edge_index2[1].astype(I32) + N1])

    E = src.shape[0]
    N = x.shape[0]

    src_a = src.reshape(E // BLK, BLK, 1)
    src_b = src.reshape(E // BLK, 1, BLK)
    b_a = batch.reshape(N // BLK, BLK, 1)
    b_b = batch.reshape(N // BLK, 1, BLK)
    cum = _graph_starts(b_a, b_b)
    gidx = (dst, src, src_a, src_b, cum, b_a, b_b)

    p = params
    x_h = _enc(x, _full_w(p['enc_x']))
    e_h = _enc(e, _full_w(p['enc_e']))
    u_h = _enc(u, _full_w(p['enc_u']))

    for _ in range(3):
        x_h, e_h, u_h, _unused = _meta(
            p['rec_e'], p['rec_x'], p['rec_u'],
            [x, x_h], [e, e_h], [u, u_h], gidx, attention=False)

    _xa, _ea, u_a, (aggxw, aggew) = _meta(
        p['att_e'], p['att_x'], p['att_u'],
        [x_h], [e_h], [u_h], gidx, attention=True)

    return _final(aggxw, aggew, u_a, u_h,
                  _full_w(p['agg_u']), _full_w(p['final']))


# speedup vs baseline: 2.7579x; 2.4138x over previous
import functools

import jax
import jax.numpy as jnp
from jax import lax
from jax.experimental import pallas as pl
from jax.experimental.pallas import tpu as pltpu
from jax.experimental.pallas import tpu_sc as plsc

F32 = jnp.float32
I32 = jnp.int32
SDS = jax.ShapeDtypeStruct
PREC = lax.Precision.DEFAULT

N1 = 10000
E1 = 160000
NG = 16
H = 128

BLK = 1000


def _dot(a, b):
    return lax.dot_general(a, b, (((a.ndim - 1,), (0,)), ((), ())),
                           precision=PREC, preferred_element_type=F32)


def _mlp23(h, l2, l3):
    (w2, b2), (w3, b3) = l2, l3
    h = jnp.maximum(_dot(h, w2[...]) + b2[...], 0.0)
    return _dot(h, w3[...]) + b3[...]


def _wspec(arr):
    nd = arr.ndim
    return pl.BlockSpec(arr.shape, lambda i: (0,) * nd)


def _full_w(params):
    return [(w, b.reshape(1, -1)) for (w, b) in params]


def _enc(x, layers):
    R = x.shape[0]
    blk = BLK if R % BLK == 0 else R
    (w1, b1), l2, l3 = layers

    def body(x_ref, w1r, b1r, w2r, b2r, w3r, b3r, o_ref):
        h = jnp.maximum(_dot(x_ref[...], w1r[...]) + b1r[...], 0.0)
        o_ref[...] = _mlp23(h, (w2r, b2r), (w3r, b3r))

    ins = [x, w1, b1, l2[0], l2[1], l3[0], l3[1]]
    specs = [pl.BlockSpec((blk, x.shape[1]), lambda i: (i, 0))] + [_wspec(a) for a in ins[1:]]
    return pl.pallas_call(
        body, grid=(R // blk,), in_specs=specs,
        out_specs=pl.BlockSpec((blk, H), lambda i: (i, 0)),
        out_shape=SDS((R, H), F32))(*ins)


def _a_proj(x_parts, w1x):
    R = x_parts[0].shape[0]
    np_ = len(x_parts)

    def body(*refs):
        xr = refs[:np_]
        wr = refs[np_]
        o_ref = refs[np_ + 1]
        xc = jnp.concatenate([r[...] for r in xr], axis=1)
        o_ref[...] = _dot(xc, wr[...])

    ins = list(x_parts) + [w1x]
    specs = [pl.BlockSpec((BLK, H), lambda i: (i, 0)) for _ in x_parts] + [_wspec(w1x)]
    return pl.pallas_call(
        body, grid=(R // BLK,), in_specs=specs,
        out_specs=pl.BlockSpec((BLK, H), lambda i: (i, 0)),
        out_shape=SDS((R, H), F32))(*ins)


def _graph_starts(b_a, b_b):
    nblk = b_a.shape[0]

    def body(bar, bbr, row_ref, col_ref):
        i = pl.program_id(0)
        ids_a = bar[...][0]
        ids_b = bbr[...][0]
        lt = (ids_a < lax.broadcasted_iota(I32, (BLK, NG), 1)).astype(F32)
        ltT = (lax.broadcasted_iota(I32, (NG, BLK), 0) > ids_b).astype(F32)

        @pl.when(i == 0)
        def _():
            row_ref[...] = jnp.zeros((1, NG), F32)
            col_ref[...] = jnp.zeros((NG, 1), F32)

        row_ref[...] += _dot(jnp.ones((1, BLK), F32), lt)
        col_ref[...] += _dot(ltT, jnp.ones((BLK, 1), F32))

    return pl.pallas_call(
        body, grid=(nblk,),
        in_specs=[pl.BlockSpec((1, BLK, 1), lambda i: (i, 0, 0)),
                  pl.BlockSpec((1, 1, BLK), lambda i: (i, 0, 0))],
        out_specs=(pl.BlockSpec((1, NG), lambda i: (0, 0)),
                   pl.BlockSpec((NG, 1), lambda i: (0, 0))),
        out_shape=(SDS((1, NG), F32), SDS((NG, 1), F32)))(b_a, b_b)


def _seg_onehots(ids_a, ids_b, cum_row, cum_col):
    fa = ids_a.astype(F32)
    fb = ids_b.astype(F32)
    ge = (fa >= cum_row).astype(F32)
    geT = (fb >= cum_col).astype(F32)
    oh = ge - jnp.concatenate([ge[:, 1:], jnp.zeros((ge.shape[0], 1), F32)], 1)
    ohT = geT - jnp.concatenate([geT[1:, :], jnp.zeros((1, geT.shape[1]), F32)], 0)
    return oh, ohT


def _edge(ad, as_, e_parts, src_a, src_b, cum, u_parts, w1e, w1u, b1, l2, l3,
          ew_mult=None):
    E = ad.shape[0]
    cum_row, cum_col = cum
    ne, nu = len(e_parts), len(u_parts)
    with_ew = ew_mult is not None

    def body(*refs):
        it = iter(refs)
        adr = next(it); asr = next(it)
        ers = [next(it) for _ in range(ne)]
        sar = next(it); sbr = next(it)
        crr = next(it); ccr = next(it)
        urs = [next(it) for _ in range(nu)]
        w1er = next(it); w1ur = next(it); b1r = next(it)
        w2r = next(it); b2r = next(it); w3r = next(it); b3r = next(it)
        mr = next(it) if with_ew else None
        e2_ref = next(it)
        aggg_ref = next(it)
        aggew_ref = next(it) if with_ew else None

        i = pl.program_id(0)
        ecat = jnp.concatenate([r[...] for r in ers], axis=1)
        ucat = jnp.concatenate([r[...] for r in urs], axis=1)
        gu = _dot(ucat, w1ur[...]) + b1r[...]
        oh, ohT = _seg_onehots(sar[...][0], sbr[...][0], crr[...], ccr[...])
        h = adr[...] - asr[...] + _dot(ecat, w1er[...]) + _dot(oh, gu)
        h = jnp.maximum(h, 0.0)
        e2 = _mlp23(h, (w2r, b2r), (w3r, b3r))
        e2_ref[...] = e2

        @pl.when(i == 0)
        def _():
            aggg_ref[...] = jnp.zeros((NG, H), F32)
            if with_ew:
                aggew_ref[...] = jnp.zeros((NG, H), F32)

        aggg_ref[...] += _dot(ohT, e2)
        if with_ew:
            aggew_ref[...] += _dot(ohT, e2 * mr[...])

    ins = [ad, as_] + list(e_parts) + [src_a, src_b, cum_row, cum_col] + \
        list(u_parts) + [w1e, w1u, b1, l2[0], l2[1], l3[0], l3[1]]
    specs = [pl.BlockSpec((BLK, H), lambda i: (i, 0)) for _ in range(2 + ne)]
    specs += [pl.BlockSpec((1, BLK, 1), lambda i: (i, 0, 0)),
              pl.BlockSpec((1, 1, BLK), lambda i: (i, 0, 0))]
    specs += [_wspec(a) for a in ins[4 + ne:]]
    if with_ew:
        ins.append(ew_mult)
        specs.append(pl.BlockSpec((BLK, H), lambda i: (i, 0)))

    out_shape = [SDS((E, H), F32), SDS((NG, H), F32)]
    out_specs = [pl.BlockSpec((BLK, H), lambda i: (i, 0)),
                 pl.BlockSpec((NG, H), lambda i: (0, 0))]
    if with_ew:
        out_shape.append(SDS((NG, H), F32))
        out_specs.append(pl.BlockSpec((NG, H), lambda i: (0, 0)))
    return pl.pallas_call(
        body, grid=(E // BLK,), in_specs=specs, out_specs=tuple(out_specs),
        out_shape=tuple(out_shape))(*ins)


def _node(x_parts, aggn, b_a, b_b, u_parts, w1xa, w1u, b1, l2, l3, xw_mult=None):
    R = aggn.shape[0]
    nx, nu = len(x_parts), len(u_parts)
    with_xw = xw_mult is not None

    def body(*refs):
        it = iter(refs)
        xrs = [next(it) for _ in range(nx)]
        aggr = next(it)
        bar = next(it); bbr = next(it)
        urs = [next(it) for _ in range(nu)]
        w1r = next(it); w1ur = next(it); b1r = next(it)
        w2r = next(it); b2r = next(it); w3r = next(it); b3r = next(it)
        mr = next(it) if with_xw else None
        x2_ref = next(it)
        aggx_ref = next(it)
        aggxw_ref = next(it) if with_xw else None

        i = pl.program_id(0)
        xc = jnp.concatenate([r[...] for r in xrs] + [aggr[...]], axis=1)
        ucat = jnp.concatenate([r[...] for r in urs], axis=1)
        gu = _dot(ucat, w1ur[...]) + b1r[...]
        ids_a = bar[...][0]
        ids_b = bbr[...][0]
        oh = (ids_a == lax.broadcasted_iota(I32, (BLK, NG), 1)).astype(F32)
        ohT = (ids_b == lax.broadcasted_iota(I32, (NG, BLK), 0)).astype(F32)
        h = jnp.maximum(_dot(xc, w1r[...]) + _dot(oh, gu), 0.0)
        x2 = _mlp23(h, (w2r, b2r), (w3r, b3r))
        x2_ref[...] = x2

        @pl.when(i == 0)
        def _():
            aggx_ref[...] = jnp.zeros((NG, H), F32)
            if with_xw:
                aggxw_ref[...] = jnp.zeros((NG, H), F32)

        aggx_ref[...] += _dot(ohT, x2)
        if with_xw:
            aggxw_ref[...] += _dot(ohT, x2 * mr[...])

    ins = list(x_parts) + [aggn, b_a, b_b] + list(u_parts) + \
        [w1xa, w1u, b1, l2[0], l2[1], l3[0], l3[1]]
    specs = [pl.BlockSpec((BLK, H), lambda i: (i, 0)) for _ in range(nx + 1)]
    specs += [pl.BlockSpec((1, BLK, 1), lambda i: (i, 0, 0)),
              pl.BlockSpec((1, 1, BLK), lambda i: (i, 0, 0))]
    specs += [_wspec(a) for a in ins[nx + 3:]]
    if with_xw:
        ins.append(xw_mult)
        specs.append(pl.BlockSpec((BLK, H), lambda i: (i, 0)))

    out_shape = [SDS((R, H), F32), SDS((NG, H), F32)]
    out_specs = [pl.BlockSpec((BLK, H), lambda i: (i, 0)),
                 pl.BlockSpec((NG, H), lambda i: (0, 0))]
    if with_xw:
        out_shape.append(SDS((NG, H), F32))
        out_specs.append(pl.BlockSpec((NG, H), lambda i: (0, 0)))
    return pl.pallas_call(
        body, grid=(R // BLK,), in_specs=specs, out_specs=tuple(out_specs),
        out_shape=tuple(out_shape))(*ins)


def _useg(aggx, aggg, u_parts, layers):
    nu = len(u_parts)
    (w1, b1), l2, l3 = layers

    def body(*refs):
        it = iter(refs)
        ax = next(it); ag = next(it)
        urs = [next(it) for _ in range(nu)]
        w1r = next(it); b1r = next(it)
        w2r = next(it); b2r = next(it); w3r = next(it); b3r = next(it)
        o_ref = next(it)
        xin = jnp.concatenate([ax[...], ag[...]] + [r[...] for r in urs], axis=1)
        h = jnp.maximum(_dot(xin, w1r[...]) + b1r[...], 0.0)
        o_ref[...] = _mlp23(h, (w2r, b2r), (w3r, b3r))

    ins = [aggx, aggg] + list(u_parts) + [w1, b1, l2[0], l2[1], l3[0], l3[1]]
    specs = [_wspec(a) for a in ins]
    return pl.pallas_call(
        body, grid=(1,), in_specs=specs,
        out_specs=pl.BlockSpec((NG, H), lambda i: (0, 0)),
        out_shape=SDS((NG, H), F32))(*ins)


def _final(aggxw, aggew, u_a, u_h, lay_agg, lay_fin):
    def body(axw, aew, uar, uhr,
             aw1, ab1, aw2, ab2, aw3, ab3,
             fw1, fb1, fw2, fb2, fw3, fb3, o_ref):
        uw = uar[...] * uhr[...]
        gin = jnp.concatenate([axw[...], aew[...], uw], axis=1)
        h = jnp.maximum(_dot(gin, aw1[...]) + ab1[...], 0.0)
        g = _mlp23(h, (aw2, ab2), (aw3, ab3))
        d = g[0:8, :] - g[8:16, :]
        h2 = jnp.maximum(_dot(d, fw1[...]) + fb1[...], 0.0)
        o_ref[...] = _mlp23(h2, (fw2, fb2), (fw3, fb3))

    (aw1, ab1), (aw2, ab2), (aw3, ab3) = lay_agg
    (fw1, fb1), (fw2, fb2), (fw3, fb3) = lay_fin
    ins = [aggxw, aggew, u_a, u_h,
           aw1, ab1, aw2, ab2, aw3, ab3, fw1, fb1, fw2, fb2, fw3, fb3]
    specs = [_wspec(a) for a in ins]
    return pl.pallas_call(
        body, grid=(1,), in_specs=specs,
        out_specs=pl.BlockSpec((8, 64), lambda i: (0, 0)),
        out_shape=SDS((8, 64), F32))(*ins)


_MESH = dict(core_axis_name="c", subcore_axis_name="s")
_NC, _NS = 2, 16
_NW = _NC * _NS


def _sc_gather2(a, dst, src):
    E = dst.shape[0]
    CH = 128
    nch = E // CH
    iters = -(-nch // _NW)

    @functools.partial(
        pl.kernel,
        out_type=(SDS((E, H), F32), SDS((E, H), F32)),
        mesh=plsc.VectorSubcoreMesh(**_MESH),
        scratch_types=[pltpu.VMEM((CH,), I32), pltpu.VMEM((CH,), I32),
                       pltpu.VMEM((CH, H), F32), pltpu.VMEM((CH, H), F32),
                       pltpu.SemaphoreType.DMA, pltpu.SemaphoreType.DMA],
    )
    def k(a_hbm, dst_hbm, src_hbm, ad_hbm, as_hbm, i1, i2, r1, r2, s1, s2):
        cid = lax.axis_index("c")
        sid = lax.axis_index("s")
        wid = sid * _NC + cid

        def body(t, carry):
            ch = wid + t * _NW

            @pl.when(ch < nch)
            def _():
                base = ch * CH
                pltpu.sync_copy(dst_hbm.at[pl.ds(base, CH)], i1)
                pltpu.sync_copy(src_hbm.at[pl.ds(base, CH)], i2)
                c1 = pltpu.async_copy(a_hbm.at[i1], r1, s1)
                c2 = pltpu.async_copy(a_hbm.at[i2], r2, s2)
                c1.wait()
                c2.wait()
                pltpu.sync_copy(r1, ad_hbm.at[pl.ds(base, CH)])
                pltpu.sync_copy(r2, as_hbm.at[pl.ds(base, CH)])

            return carry

        lax.fori_loop(0, iters, body, 0)

    return k(a, dst, src)


def _sc_scatter(e2, dst):
    E = dst.shape[0]
    CH = 128
    epc = E // _NC
    nch = epc // CH
    iters = -(-nch // _NS)
    HALF = N1 // 2
    DUMMY = HALF
    rsmall, rbig = 312, 320

    @functools.partial(
        pl.kernel,
        out_type=SDS((2 * N1, H), F32),
        mesh=plsc.VectorSubcoreMesh(**_MESH),
        scratch_types=[pltpu.VMEM((CH,), I32),
                       pltpu.VMEM((CH, H), F32),
                       pltpu.VMEM((rbig, H), F32),
                       pltpu.VMEM_SHARED((HALF + 120, H), F32)],
    )
    def k(e2_hbm, dst_hbm, aggn_hbm, ibuf, rbuf, zbuf, acc):
        cid = lax.axis_index("c")
        sid = lax.axis_index("s")

        def zrow(r, carry):
            for j in range(H // 16):
                zbuf[r, pl.ds(j * 16, 16)] = jnp.zeros((16,), F32)
            return carry

        lax.fori_loop(0, rbig, zrow, 0)
        rbase = sid * rsmall
        last = _NS - 1

        for h in range(2):
            lo = cid * N1 + h * HALF

            @pl.when(sid < last)
            def _():
                pltpu.sync_copy(zbuf.at[pl.ds(0, rsmall)],
                                acc.at[pl.ds(rbase, rsmall)])

            @pl.when(sid == last)
            def _():
                pltpu.sync_copy(zbuf, acc.at[pl.ds(last * rsmall, rbig)])

            plsc.subcore_barrier()

            def body(t, carry):
                ch = sid + t * _NS

                @pl.when(ch < nch)
                def _():
                    base = cid * epc + ch * CH
                    pltpu.sync_copy(e2_hbm.at[pl.ds(base, CH)], rbuf)
                    pltpu.sync_copy(dst_hbm.at[pl.ds(base, CH)], ibuf)
                    for j in range(CH // 16):
                        v = ibuf[pl.ds(j * 16, 16)] - lo
                        ok = (v >= 0) & (v < HALF)
                        ibuf[pl.ds(j * 16, 16)] = jnp.where(ok, v, DUMMY)
                    pltpu.sync_copy(rbuf, acc.at[ibuf], add=True)

                return carry

            lax.fori_loop(0, iters, body, 0)
            plsc.subcore_barrier()

            @pl.when(sid < last)
            def _():
                pltpu.sync_copy(acc.at[pl.ds(rbase, rsmall)],
                                aggn_hbm.at[pl.ds(lo + rbase, rsmall)])

            @pl.when(sid == last)
            def _():
                pltpu.sync_copy(acc.at[pl.ds(last * rsmall, rbig)],
                                aggn_hbm.at[pl.ds(lo + last * rsmall, rbig)])

            plsc.subcore_barrier()

    return k(e2, dst)


def _meta(pe, px, pu, x_parts, e_parts, u_parts, gidx, attention=False):
    dst, src, src_a, src_b, cum, b_a, b_b = gidx
    npart = len(x_parts)
    K = npart * H

    (we1, be1), el2, el3 = _full_w(pe)
    (wx1, bx1), xl2, xl3 = _full_w(px)

    w1x = we1[0:K]
    w1e = we1[K:2 * K]
    w1u = we1[2 * K:3 * K]
    a = _a_proj(x_parts, w1x)
    ad, as_ = _sc_gather2(a, dst, src)

    ew_mult = e_parts[0] if attention else None
    eouts = _edge(ad, as_, e_parts, src_a, src_b, cum, u_parts,
                  w1e, w1u, be1, el2, el3, ew_mult=ew_mult)
    if attention:
        e2, aggg, aggew = eouts
    else:
        e2, aggg = eouts
        aggew = None

    aggn = _sc_scatter(e2, dst)

    w1xa = wx1[0:K + H]
    w1xu = wx1[K + H:K + H + npart * H]
    xw_mult = x_parts[0] if attention else None
    nouts = _node(x_parts, aggn, b_a, b_b, u_parts,
                  w1xa, w1xu, bx1, xl2, xl3, xw_mult=xw_mult)
    if attention:
        x2, aggx, aggxw = nouts
    else:
        x2, aggx = nouts
        aggxw = None

    u2 = _useg(aggx, aggg, u_parts, _full_w(pu))
    return x2, e2, u2, (aggxw, aggew)


def kernel(x1, edge_index1, e1, u1, batch1, x2, edge_index2, e2, u2, batch2, params):
    x = jnp.concatenate([x1, x2], axis=0)
    e = jnp.concatenate([e1, e2], axis=0)
    u = jnp.concatenate([u1, u2], axis=0)
    batch = jnp.concatenate(
        [batch1.astype(I32), batch2.astype(I32) + 8], axis=0)
    src = jnp.concatenate(
        [edge_index1[0].astype(I32), edge_index2[0].astype(I32) + N1])
    dst = jnp.concatenate(
        [edge_index1[1].astype(I32), edge_index2[1].astype(I32) + N1])

    E = src.shape[0]
    N = x.shape[0]

    src_a = src.reshape(E // BLK, BLK, 1)
    src_b = src.reshape(E // BLK, 1, BLK)
    b_a = batch.reshape(N // BLK, BLK, 1)
    b_b = batch.reshape(N // BLK, 1, BLK)
    cum = _graph_starts(b_a, b_b)
    gidx = (dst, src, src_a, src_b, cum, b_a, b_b)

    p = params
    x_h = _enc(x, _full_w(p['enc_x']))
    e_h = _enc(e, _full_w(p['enc_e']))
    u_h = _enc(u, _full_w(p['enc_u']))

    for _ in range(3):
        x_h, e_h, u_h, _unused = _meta(
            p['rec_e'], p['rec_x'], p['rec_u'],
            [x, x_h], [e, e_h], [u, u_h], gidx, attention=False)

    _xa, _ea, u_a, (aggxw, aggew) = _meta(
        p['att_e'], p['att_x'], p['att_u'],
        [x_h], [e_h], [u_h], gidx, attention=True)

    return _final(aggxw, aggew, u_a, u_h,
                  _full_w(p['agg_u']), _full_w(p['final']))


# speedup vs baseline: 3.0109x; 1.0917x over previous
import functools

import jax
import jax.numpy as jnp
from jax import lax
from jax.experimental import pallas as pl
from jax.experimental.pallas import tpu as pltpu
from jax.experimental.pallas import tpu_sc as plsc

F32 = jnp.float32
I32 = jnp.int32
SDS = jax.ShapeDtypeStruct
PREC = lax.Precision.DEFAULT

N1 = 10000
E1 = 160000
NG = 16
H = 128

BLK = 1000


def _dot(a, b):
    return lax.dot_general(a, b, (((a.ndim - 1,), (0,)), ((), ())),
                           precision=PREC, preferred_element_type=F32)


def _mlp23(h, l2, l3):
    (w2, b2), (w3, b3) = l2, l3
    h = jnp.maximum(_dot(h, w2[...]) + b2[...], 0.0)
    return _dot(h, w3[...]) + b3[...]


def _wspec(arr):
    nd = arr.ndim
    return pl.BlockSpec(arr.shape, lambda i: (0,) * nd)


def _full_w(params):
    return [(w, b.reshape(1, -1)) for (w, b) in params]


def _enc(x, layers):
    R = x.shape[0]
    blk = BLK if R % BLK == 0 else R
    (w1, b1), l2, l3 = layers

    def body(x_ref, w1r, b1r, w2r, b2r, w3r, b3r, o_ref):
        h = jnp.maximum(_dot(x_ref[...], w1r[...]) + b1r[...], 0.0)
        o_ref[...] = _mlp23(h, (w2r, b2r), (w3r, b3r))

    ins = [x, w1, b1, l2[0], l2[1], l3[0], l3[1]]
    specs = [pl.BlockSpec((blk, x.shape[1]), lambda i: (i, 0))] + [_wspec(a) for a in ins[1:]]
    return pl.pallas_call(
        body, grid=(R // blk,), in_specs=specs,
        out_specs=pl.BlockSpec((blk, H), lambda i: (i, 0)),
        out_shape=SDS((R, H), F32))(*ins)


def _a_proj(x_parts, w1x):
    R = x_parts[0].shape[0]
    np_ = len(x_parts)

    def body(*refs):
        xr = refs[:np_]
        wr = refs[np_]
        o_ref = refs[np_ + 1]
        xc = jnp.concatenate([r[...] for r in xr], axis=1)
        o_ref[...] = _dot(xc, wr[...])

    ins = list(x_parts) + [w1x]
    specs = [pl.BlockSpec((BLK, H), lambda i: (i, 0)) for _ in x_parts] + [_wspec(w1x)]
    return pl.pallas_call(
        body, grid=(R // BLK,), in_specs=specs,
        out_specs=pl.BlockSpec((BLK, H), lambda i: (i, 0)),
        out_shape=SDS((R, H), F32))(*ins)


def _graph_starts(b_a, b_b):
    nblk = b_a.shape[0]

    def body(bar, bbr, row_ref, col_ref):
        i = pl.program_id(0)
        ids_a = bar[...][0]
        ids_b = bbr[...][0]
        lt = (ids_a < lax.broadcasted_iota(I32, (BLK, NG), 1)).astype(F32)
        ltT = (lax.broadcasted_iota(I32, (NG, BLK), 0) > ids_b).astype(F32)

        @pl.when(i == 0)
        def _():
            row_ref[...] = jnp.zeros((1, NG), F32)
            col_ref[...] = jnp.zeros((NG, 1), F32)

        row_ref[...] += _dot(jnp.ones((1, BLK), F32), lt)
        col_ref[...] += _dot(ltT, jnp.ones((BLK, 1), F32))

    return pl.pallas_call(
        body, grid=(nblk,),
        in_specs=[pl.BlockSpec((1, BLK, 1), lambda i: (i, 0, 0)),
                  pl.BlockSpec((1, 1, BLK), lambda i: (i, 0, 0))],
        out_specs=(pl.BlockSpec((1, NG), lambda i: (0, 0)),
                   pl.BlockSpec((NG, 1), lambda i: (0, 0))),
        out_shape=(SDS((1, NG), F32), SDS((NG, 1), F32)))(b_a, b_b)


def _seg_onehots(ids_a, ids_b, cum_row, cum_col):
    fa = ids_a.astype(F32)
    fb = ids_b.astype(F32)
    ge = (fa >= cum_row).astype(F32)
    geT = (fb >= cum_col).astype(F32)
    oh = ge - jnp.concatenate([ge[:, 1:], jnp.zeros((ge.shape[0], 1), F32)], 1)
    ohT = geT - jnp.concatenate([geT[1:, :], jnp.zeros((1, geT.shape[1]), F32)], 0)
    return oh, ohT


def _edge(d, e_parts, src_a, src_b, cum, u_parts, w1e, w1u, b1, l2, l3,
          ew_mult=None):
    E = d.shape[0]
    cum_row, cum_col = cum
    ne, nu = len(e_parts), len(u_parts)
    with_ew = ew_mult is not None

    def body(*refs):
        it = iter(refs)
        dr = next(it)
        ers = [next(it) for _ in range(ne)]
        sar = next(it); sbr = next(it)
        crr = next(it); ccr = next(it)
        urs = [next(it) for _ in range(nu)]
        w1er = next(it); w1ur = next(it); b1r = next(it)
        w2r = next(it); b2r = next(it); w3r = next(it); b3r = next(it)
        mr = next(it) if with_ew else None
        e2_ref = next(it)
        aggg_ref = next(it)
        aggew_ref = next(it) if with_ew else None

        i = pl.program_id(0)
        ecat = jnp.concatenate([r[...] for r in ers], axis=1)
        ucat = jnp.concatenate([r[...] for r in urs], axis=1)
        gu = _dot(ucat, w1ur[...]) + b1r[...]
        oh, ohT = _seg_onehots(sar[...][0], sbr[...][0], crr[...], ccr[...])
        h = dr[...] + _dot(ecat, w1er[...]) + _dot(oh, gu)
        h = jnp.maximum(h, 0.0)
        e2 = _mlp23(h, (w2r, b2r), (w3r, b3r))
        e2_ref[...] = e2

        @pl.when(i == 0)
        def _():
            aggg_ref[...] = jnp.zeros((NG, H), F32)
            if with_ew:
                aggew_ref[...] = jnp.zeros((NG, H), F32)

        aggg_ref[...] += _dot(ohT, e2)
        if with_ew:
            aggew_ref[...] += _dot(ohT, e2 * mr[...])

    ins = [d] + list(e_parts) + [src_a, src_b, cum_row, cum_col] + \
        list(u_parts) + [w1e, w1u, b1, l2[0], l2[1], l3[0], l3[1]]
    specs = [pl.BlockSpec((BLK, H), lambda i: (i, 0)) for _ in range(1 + ne)]
    specs += [pl.BlockSpec((1, BLK, 1), lambda i: (i, 0, 0)),
              pl.BlockSpec((1, 1, BLK), lambda i: (i, 0, 0))]
    specs += [_wspec(a) for a in ins[3 + ne:]]
    if with_ew:
        ins.append(ew_mult)
        specs.append(pl.BlockSpec((BLK, H), lambda i: (i, 0)))

    out_shape = [SDS((E, H), F32), SDS((NG, H), F32)]
    out_specs = [pl.BlockSpec((BLK, H), lambda i: (i, 0)),
                 pl.BlockSpec((NG, H), lambda i: (0, 0))]
    if with_ew:
        out_shape.append(SDS((NG, H), F32))
        out_specs.append(pl.BlockSpec((NG, H), lambda i: (0, 0)))
    return pl.pallas_call(
        body, grid=(E // BLK,), in_specs=specs, out_specs=tuple(out_specs),
        out_shape=tuple(out_shape))(*ins)


def _node(x_parts, aggn, b_a, b_b, u_parts, w1xa, w1u, b1, l2, l3, xw_mult=None):
    R = aggn.shape[0]
    nx, nu = len(x_parts), len(u_parts)
    with_xw = xw_mult is not None

    def body(*refs):
        it = iter(refs)
        xrs = [next(it) for _ in range(nx)]
        aggr = next(it)
        bar = next(it); bbr = next(it)
        urs = [next(it) for _ in range(nu)]
        w1r = next(it); w1ur = next(it); b1r = next(it)
        w2r = next(it); b2r = next(it); w3r = next(it); b3r = next(it)
        mr = next(it) if with_xw else None
        x2_ref = next(it)
        aggx_ref = next(it)
        aggxw_ref = next(it) if with_xw else None

        i = pl.program_id(0)
        xc = jnp.concatenate([r[...] for r in xrs] + [aggr[...]], axis=1)
        ucat = jnp.concatenate([r[...] for r in urs], axis=1)
        gu = _dot(ucat, w1ur[...]) + b1r[...]
        ids_a = bar[...][0]
        ids_b = bbr[...][0]
        oh = (ids_a == lax.broadcasted_iota(I32, (BLK, NG), 1)).astype(F32)
        ohT = (ids_b == lax.broadcasted_iota(I32, (NG, BLK), 0)).astype(F32)
        h = jnp.maximum(_dot(xc, w1r[...]) + _dot(oh, gu), 0.0)
        x2 = _mlp23(h, (w2r, b2r), (w3r, b3r))
        x2_ref[...] = x2

        @pl.when(i == 0)
        def _():
            aggx_ref[...] = jnp.zeros((NG, H), F32)
            if with_xw:
                aggxw_ref[...] = jnp.zeros((NG, H), F32)

        aggx_ref[...] += _dot(ohT, x2)
        if with_xw:
            aggxw_ref[...] += _dot(ohT, x2 * mr[...])

    ins = list(x_parts) + [aggn, b_a, b_b] + list(u_parts) + \
        [w1xa, w1u, b1, l2[0], l2[1], l3[0], l3[1]]
    specs = [pl.BlockSpec((BLK, H), lambda i: (i, 0)) for _ in range(nx + 1)]
    specs += [pl.BlockSpec((1, BLK, 1), lambda i: (i, 0, 0)),
              pl.BlockSpec((1, 1, BLK), lambda i: (i, 0, 0))]
    specs += [_wspec(a) for a in ins[nx + 3:]]
    if with_xw:
        ins.append(xw_mult)
        specs.append(pl.BlockSpec((BLK, H), lambda i: (i, 0)))

    out_shape = [SDS((R, H), F32), SDS((NG, H), F32)]
    out_specs = [pl.BlockSpec((BLK, H), lambda i: (i, 0)),
                 pl.BlockSpec((NG, H), lambda i: (0, 0))]
    if with_xw:
        out_shape.append(SDS((NG, H), F32))
        out_specs.append(pl.BlockSpec((NG, H), lambda i: (0, 0)))
    return pl.pallas_call(
        body, grid=(R // BLK,), in_specs=specs, out_specs=tuple(out_specs),
        out_shape=tuple(out_shape))(*ins)


def _useg(aggx, aggg, u_parts, layers):
    nu = len(u_parts)
    (w1, b1), l2, l3 = layers

    def body(*refs):
        it = iter(refs)
        ax = next(it); ag = next(it)
        urs = [next(it) for _ in range(nu)]
        w1r = next(it); b1r = next(it)
        w2r = next(it); b2r = next(it); w3r = next(it); b3r = next(it)
        o_ref = next(it)
        xin = jnp.concatenate([ax[...], ag[...]] + [r[...] for r in urs], axis=1)
        h = jnp.maximum(_dot(xin, w1r[...]) + b1r[...], 0.0)
        o_ref[...] = _mlp23(h, (w2r, b2r), (w3r, b3r))

    ins = [aggx, aggg] + list(u_parts) + [w1, b1, l2[0], l2[1], l3[0], l3[1]]
    specs = [_wspec(a) for a in ins]
    return pl.pallas_call(
        body, grid=(1,), in_specs=specs,
        out_specs=pl.BlockSpec((NG, H), lambda i: (0, 0)),
        out_shape=SDS((NG, H), F32))(*ins)


def _final(aggxw, aggew, u_a, u_h, lay_agg, lay_fin):
    def body(axw, aew, uar, uhr,
             aw1, ab1, aw2, ab2, aw3, ab3,
             fw1, fb1, fw2, fb2, fw3, fb3, o_ref):
        uw = uar[...] * uhr[...]
        gin = jnp.concatenate([axw[...], aew[...], uw], axis=1)
        h = jnp.maximum(_dot(gin, aw1[...]) + ab1[...], 0.0)
        g = _mlp23(h, (aw2, ab2), (aw3, ab3))
        d = g[0:8, :] - g[8:16, :]
        h2 = jnp.maximum(_dot(d, fw1[...]) + fb1[...], 0.0)
        o_ref[...] = _mlp23(h2, (fw2, fb2), (fw3, fb3))

    (aw1, ab1), (aw2, ab2), (aw3, ab3) = lay_agg
    (fw1, fb1), (fw2, fb2), (fw3, fb3) = lay_fin
    ins = [aggxw, aggew, u_a, u_h,
           aw1, ab1, aw2, ab2, aw3, ab3, fw1, fb1, fw2, fb2, fw3, fb3]
    specs = [_wspec(a) for a in ins]
    return pl.pallas_call(
        body, grid=(1,), in_specs=specs,
        out_specs=pl.BlockSpec((8, 64), lambda i: (0, 0)),
        out_shape=SDS((8, 64), F32))(*ins)


_MESH = dict(core_axis_name="c", subcore_axis_name="s")
_NC, _NS = 2, 16
_NW = _NC * _NS


def _sc_gather_diff(a, dst, src):
    E = dst.shape[0]
    CH = 80
    per_w = E // _NW
    nch = per_w // CH
    npairs = nch // 2

    @functools.partial(
        pl.kernel,
        out_type=SDS((E, H), F32),
        mesh=plsc.VectorSubcoreMesh(**_MESH),
        scratch_types=[pltpu.VMEM((per_w,), I32), pltpu.VMEM((per_w,), I32),
                       pltpu.VMEM((CH, H), F32), pltpu.VMEM((CH, H), F32),
                       pltpu.VMEM((CH, H), F32),
                       pltpu.VMEM((CH, H), F32), pltpu.VMEM((CH, H), F32),
                       pltpu.VMEM((CH, H), F32),
                       pltpu.SemaphoreType.DMA, pltpu.SemaphoreType.DMA,
                       pltpu.SemaphoreType.DMA, pltpu.SemaphoreType.DMA],
    )
    def k(a_hbm, dst_hbm, src_hbm, d_hbm,
          idx_d, idx_s, r1a, r2a, wa, r1b, r2b, wb, gsa, gsb, wsa, wsb):
        cid = lax.axis_index("c")
        sid = lax.axis_index("s")
        wid = sid * _NC + cid
        ebase = wid * per_w
        pltpu.sync_copy(dst_hbm.at[pl.ds(ebase, per_w)], idx_d)
        pltpu.sync_copy(src_hbm.at[pl.ds(ebase, per_w)], idx_s)

        def fire(ch, r1, r2, gs):
            off = ch * CH
            c1 = pltpu.async_copy(a_hbm.at[idx_d.at[pl.ds(off, CH)]], r1, gs)
            c2 = pltpu.async_copy(a_hbm.at[idx_s.at[pl.ds(off, CH)]], r2, gs)
            return c1, c2

        def sub(r1, r2, w):
            def row(r, carry):
                for j in range(H // 16):
                    s = pl.ds(j * 16, 16)
                    w[r, s] = r1[r, s] - r2[r, s]
                return carry

            lax.fori_loop(0, CH, row, 0)

        def wwait(w, ws):
            pltpu.make_async_copy(w, d_hbm.at[pl.ds(0, CH)], ws).wait()

        def proc(k_, ch, descs, r1, r2, w, ws):
            descs[0].wait()
            descs[1].wait()

            @pl.when(k_ > 0)
            def _():
                wwait(w, ws)

            sub(r1, r2, w)
            pltpu.async_copy(w, d_hbm.at[pl.ds(ebase + ch * CH, CH)], ws)

        def body(t, carry):
            ca = 2 * t
            cb = 2 * t + 1
            da = fire(ca, r1a, r2a, gsa)
            db = fire(cb, r1b, r2b, gsb)
            proc(t, ca, da, r1a, r2a, wa, wsa)
            proc(t, cb, db, r1b, r2b, wb, wsb)
            return carry

        lax.fori_loop(0, npairs, body, 0)
        dl = fire(nch - 1, r1a, r2a, gsa)
        proc(jnp.int32(1), nch - 1, dl, r1a, r2a, wa, wsa)
        wwait(wa, wsa)
        wwait(wb, wsb)

    return k(a, dst, src)


def _sc_scatter(e2, dst):
    E = dst.shape[0]
    CH = 128
    epc = E // _NC
    nch = epc // CH
    iters = -(-nch // _NS)
    HALF = N1 // 2
    DUMMY = HALF
    rsmall, rbig = 312, 320

    @functools.partial(
        pl.kernel,
        out_type=SDS((2 * N1, H), F32),
        mesh=plsc.VectorSubcoreMesh(**_MESH),
        scratch_types=[pltpu.VMEM((CH,), I32),
                       pltpu.VMEM((CH, H), F32),
                       pltpu.VMEM((rbig, H), F32),
                       pltpu.VMEM_SHARED((HALF + 120, H), F32)],
    )
    def k(e2_hbm, dst_hbm, aggn_hbm, ibuf, rbuf, zbuf, acc):
        cid = lax.axis_index("c")
        sid = lax.axis_index("s")

        def zrow(r, carry):
            for j in range(H // 16):
                zbuf[r, pl.ds(j * 16, 16)] = jnp.zeros((16,), F32)
            return carry

        lax.fori_loop(0, rbig, zrow, 0)
        rbase = sid * rsmall
        last = _NS - 1

        for h in range(2):
            lo = cid * N1 + h * HALF

            @pl.when(sid < last)
            def _():
                pltpu.sync_copy(zbuf.at[pl.ds(0, rsmall)],
                                acc.at[pl.ds(rbase, rsmall)])

            @pl.when(sid == last)
            def _():
                pltpu.sync_copy(zbuf, acc.at[pl.ds(last * rsmall, rbig)])

            plsc.subcore_barrier()

            def body(t, carry):
                ch = sid + t * _NS

                @pl.when(ch < nch)
                def _():
                    base = cid * epc + ch * CH
                    pltpu.sync_copy(e2_hbm.at[pl.ds(base, CH)], rbuf)
                    pltpu.sync_copy(dst_hbm.at[pl.ds(base, CH)], ibuf)
                    for j in range(CH // 16):
                        v = ibuf[pl.ds(j * 16, 16)] - lo
                        ok = (v >= 0) & (v < HALF)
                        ibuf[pl.ds(j * 16, 16)] = jnp.where(ok, v, DUMMY)
                    pltpu.sync_copy(rbuf, acc.at[ibuf], add=True)

                return carry

            lax.fori_loop(0, iters, body, 0)
            plsc.subcore_barrier()

            @pl.when(sid < last)
            def _():
                pltpu.sync_copy(acc.at[pl.ds(rbase, rsmall)],
                                aggn_hbm.at[pl.ds(lo + rbase, rsmall)])

            @pl.when(sid == last)
            def _():
                pltpu.sync_copy(acc.at[pl.ds(last * rsmall, rbig)],
                                aggn_hbm.at[pl.ds(lo + last * rsmall, rbig)])

            plsc.subcore_barrier()

    return k(e2, dst)


def _meta(pe, px, pu, x_parts, e_parts, u_parts, gidx, attention=False):
    dst, src, src_a, src_b, cum, b_a, b_b = gidx
    npart = len(x_parts)
    K = npart * H

    (we1, be1), el2, el3 = _full_w(pe)
    (wx1, bx1), xl2, xl3 = _full_w(px)

    w1x = we1[0:K]
    w1e = we1[K:2 * K]
    w1u = we1[2 * K:3 * K]
    a = _a_proj(x_parts, w1x)
    d = _sc_gather_diff(a, dst, src)

    ew_mult = e_parts[0] if attention else None
    eouts = _edge(d, e_parts, src_a, src_b, cum, u_parts,
                  w1e, w1u, be1, el2, el3, ew_mult=ew_mult)
    if attention:
        e2, aggg, aggew = eouts
    else:
        e2, aggg = eouts
        aggew = None

    aggn = _sc_scatter(e2, dst)

    w1xa = wx1[0:K + H]
    w1xu = wx1[K + H:K + H + npart * H]
    xw_mult = x_parts[0] if attention else None
    nouts = _node(x_parts, aggn, b_a, b_b, u_parts,
                  w1xa, w1xu, bx1, xl2, xl3, xw_mult=xw_mult)
    if attention:
        x2, aggx, aggxw = nouts
    else:
        x2, aggx = nouts
        aggxw = None

    u2 = _useg(aggx, aggg, u_parts, _full_w(pu))
    return x2, e2, u2, (aggxw, aggew)


def kernel(x1, edge_index1, e1, u1, batch1, x2, edge_index2, e2, u2, batch2, params):
    x = jnp.concatenate([x1, x2], axis=0)
    e = jnp.concatenate([e1, e2], axis=0)
    u = jnp.concatenate([u1, u2], axis=0)
    batch = jnp.concatenate(
        [batch1.astype(I32), batch2.astype(I32) + 8], axis=0)
    src = jnp.concatenate(
        [edge_index1[0].astype(I32), edge_index2[0].astype(I32) + N1])
    dst = jnp.concatenate(
        [edge_index1[1].astype(I32), edge_index2[1].astype(I32) + N1])

    E = src.shape[0]
    N = x.shape[0]

    src_a = src.reshape(E // BLK, BLK, 1)
    src_b = src.reshape(E // BLK, 1, BLK)
    b_a = batch.reshape(N // BLK, BLK, 1)
    b_b = batch.reshape(N // BLK, 1, BLK)
    cum = _graph_starts(b_a, b_b)
    gidx = (dst, src, src_a, src_b, cum, b_a, b_b)

    p = params
    x_h = _enc(x, _full_w(p['enc_x']))
    e_h = _enc(e, _full_w(p['enc_e']))
    u_h = _enc(u, _full_w(p['enc_u']))

    for _ in range(3):
        x_h, e_h, u_h, _unused = _meta(
            p['rec_e'], p['rec_x'], p['rec_u'],
            [x, x_h], [e, e_h], [u, u_h], gidx, attention=False)

    _xa, _ea, u_a, (aggxw, aggew) = _meta(
        p['att_e'], p['att_x'], p['att_u'],
        [x_h], [e_h], [u_h], gidx, attention=True)

    return _final(aggxw, aggew, u_a, u_h,
                  _full_w(p['agg_u']), _full_w(p['final']))


# speedup vs baseline: 3.6804x; 1.2224x over previous
import functools

import jax
import jax.numpy as jnp
from jax import lax
from jax.experimental import pallas as pl
from jax.experimental.pallas import tpu as pltpu
from jax.experimental.pallas import tpu_sc as plsc

F32 = jnp.float32
I32 = jnp.int32
SDS = jax.ShapeDtypeStruct
PREC = lax.Precision.DEFAULT

N1 = 10000
E1 = 160000
NG = 16
H = 128

BLK = 1000


def _dot(a, b):
    return lax.dot_general(a.astype(jnp.bfloat16), b.astype(jnp.bfloat16),
                           (((a.ndim - 1,), (0,)), ((), ())),
                           precision=PREC, preferred_element_type=F32)


def _mlp23(h, l2, l3):
    (w2, b2), (w3, b3) = l2, l3
    h = jnp.maximum(_dot(h, w2[...]) + b2[...], 0.0)
    return _dot(h, w3[...]) + b3[...]


def _wspec(arr):
    nd = arr.ndim
    return pl.BlockSpec(arr.shape, lambda i: (0,) * nd)


def _full_w(params):
    return [(w, b.reshape(1, -1)) for (w, b) in params]


def _enc(x, layers):
    R = x.shape[0]
    blk = BLK if R % BLK == 0 else R
    (w1, b1), l2, l3 = layers

    def body(x_ref, w1r, b1r, w2r, b2r, w3r, b3r, o_ref):
        h = jnp.maximum(_dot(x_ref[...], w1r[...]) + b1r[...], 0.0)
        o_ref[...] = _mlp23(h, (w2r, b2r), (w3r, b3r))

    ins = [x, w1, b1, l2[0], l2[1], l3[0], l3[1]]
    specs = [pl.BlockSpec((blk, x.shape[1]), lambda i: (i, 0))] + [_wspec(a) for a in ins[1:]]
    return pl.pallas_call(
        body, grid=(R // blk,), in_specs=specs,
        out_specs=pl.BlockSpec((blk, H), lambda i: (i, 0)),
        out_shape=SDS((R, H), F32))(*ins)


def _a_proj(x_parts, w1x):
    R = x_parts[0].shape[0]
    np_ = len(x_parts)

    def body(*refs):
        xr = refs[:np_]
        wr = refs[np_]
        o_ref = refs[np_ + 1]
        xc = jnp.concatenate([r[...] for r in xr], axis=1)
        o_ref[...] = _dot(xc, wr[...])

    ins = list(x_parts) + [w1x]
    specs = [pl.BlockSpec((BLK, H), lambda i: (i, 0)) for _ in x_parts] + [_wspec(w1x)]
    return pl.pallas_call(
        body, grid=(R // BLK,), in_specs=specs,
        out_specs=pl.BlockSpec((BLK, H), lambda i: (i, 0)),
        out_shape=SDS((R, H), F32))(*ins)


def _graph_starts(b_a, b_b):
    nblk = b_a.shape[0]

    def body(bar, bbr, row_ref, col_ref):
        i = pl.program_id(0)
        ids_a = bar[...][0]
        ids_b = bbr[...][0]
        lt = (ids_a < lax.broadcasted_iota(I32, (BLK, NG), 1)).astype(F32)
        ltT = (lax.broadcasted_iota(I32, (NG, BLK), 0) > ids_b).astype(F32)

        @pl.when(i == 0)
        def _():
            row_ref[...] = jnp.zeros((1, NG), F32)
            col_ref[...] = jnp.zeros((NG, 1), F32)

        row_ref[...] += _dot(jnp.ones((1, BLK), F32), lt)
        col_ref[...] += _dot(ltT, jnp.ones((BLK, 1), F32))

    return pl.pallas_call(
        body, grid=(nblk,),
        in_specs=[pl.BlockSpec((1, BLK, 1), lambda i: (i, 0, 0)),
                  pl.BlockSpec((1, 1, BLK), lambda i: (i, 0, 0))],
        out_specs=(pl.BlockSpec((1, NG), lambda i: (0, 0)),
                   pl.BlockSpec((NG, 1), lambda i: (0, 0))),
        out_shape=(SDS((1, NG), F32), SDS((NG, 1), F32)))(b_a, b_b)


def _seg_onehots(ids_a, ids_b, cum_row, cum_col):
    fa = ids_a.astype(F32)
    fb = ids_b.astype(F32)
    ge = (fa >= cum_row).astype(F32)
    geT = (fb >= cum_col).astype(F32)
    oh = ge - jnp.concatenate([ge[:, 1:], jnp.zeros((ge.shape[0], 1), F32)], 1)
    ohT = geT - jnp.concatenate([geT[1:, :], jnp.zeros((1, geT.shape[1]), F32)], 0)
    return oh, ohT


def _edge(d, e_parts, src_a, src_b, cum, u_parts, w1e, w1u, b1, l2, l3,
          with_ew=False):
    E = d.shape[0]
    cum_row, cum_col = cum
    ne, nu = len(e_parts), len(u_parts)

    def body(*refs):
        it = iter(refs)
        dr = next(it)
        ers = [next(it) for _ in range(ne)]
        sar = next(it); sbr = next(it)
        crr = next(it); ccr = next(it)
        urs = [next(it) for _ in range(nu)]
        w1er = next(it); w1ur = next(it); b1r = next(it)
        w2r = next(it); b2r = next(it); w3r = next(it); b3r = next(it)
        e2_ref = next(it)
        aggg_ref = next(it)
        aggew_ref = next(it) if with_ew else None

        i = pl.program_id(0)
        ecat = jnp.concatenate([r[...] for r in ers], axis=1)
        ucat = jnp.concatenate([r[...] for r in urs], axis=1)
        gu = _dot(ucat, w1ur[...]) + b1r[...]
        oh, ohT = _seg_onehots(sar[...][0], sbr[...][0], crr[...], ccr[...])
        h = dr[...] + _dot(ecat, w1er[...]) + _dot(oh, gu)
        h = jnp.maximum(h, 0.0)
        e2 = _mlp23(h, (w2r, b2r), (w3r, b3r))
        e2_ref[...] = e2

        @pl.when(i == 0)
        def _():
            aggg_ref[...] = jnp.zeros((NG, H), F32)
            if with_ew:
                aggew_ref[...] = jnp.zeros((NG, H), F32)

        aggg_ref[...] += _dot(ohT, e2)
        if with_ew:
            aggew_ref[...] += _dot(ohT, e2 * ecat)

    ins = [d] + list(e_parts) + [src_a, src_b, cum_row, cum_col] + \
        list(u_parts) + [w1e, w1u, b1, l2[0], l2[1], l3[0], l3[1]]
    specs = [pl.BlockSpec((BLK, H), lambda i: (i, 0))]
    specs += [pl.BlockSpec((BLK, p.shape[1]), lambda i: (i, 0)) for p in e_parts]
    specs += [pl.BlockSpec((1, BLK, 1), lambda i: (i, 0, 0)),
              pl.BlockSpec((1, 1, BLK), lambda i: (i, 0, 0))]
    specs += [_wspec(a) for a in ins[3 + ne:]]

    out_shape = [SDS((E, H), F32), SDS((NG, H), F32)]
    out_specs = [pl.BlockSpec((BLK, H), lambda i: (i, 0)),
                 pl.BlockSpec((NG, H), lambda i: (0, 0))]
    if with_ew:
        out_shape.append(SDS((NG, H), F32))
        out_specs.append(pl.BlockSpec((NG, H), lambda i: (0, 0)))
    return pl.pallas_call(
        body, grid=(E // BLK,), in_specs=specs, out_specs=tuple(out_specs),
        out_shape=tuple(out_shape))(*ins)


def _node(x_parts, aggn_parts, b_a, b_b, u_parts, w1xa, w1u, b1, l2, l3,
          with_xw=False):
    R = x_parts[0].shape[0]
    nx, na, nu = len(x_parts), len(aggn_parts), len(u_parts)

    def body(*refs):
        it = iter(refs)
        xrs = [next(it) for _ in range(nx)]
        ars = [next(it) for _ in range(na)]
        bar = next(it); bbr = next(it)
        urs = [next(it) for _ in range(nu)]
        w1r = next(it); w1ur = next(it); b1r = next(it)
        w2r = next(it); b2r = next(it); w3r = next(it); b3r = next(it)
        x2_ref = next(it)
        aggx_ref = next(it)
        aggxw_ref = next(it) if with_xw else None

        i = pl.program_id(0)
        xpcat = jnp.concatenate([r[...] for r in xrs], axis=1)
        xc = jnp.concatenate([xpcat] + [r[...] for r in ars], axis=1)
        ucat = jnp.concatenate([r[...] for r in urs], axis=1)
        gu = _dot(ucat, w1ur[...]) + b1r[...]
        ids_a = bar[...][0]
        ids_b = bbr[...][0]
        oh = (ids_a == lax.broadcasted_iota(I32, (BLK, NG), 1)).astype(F32)
        ohT = (ids_b == lax.broadcasted_iota(I32, (NG, BLK), 0)).astype(F32)
        h = jnp.maximum(_dot(xc, w1r[...]) + _dot(oh, gu), 0.0)
        x2 = _mlp23(h, (w2r, b2r), (w3r, b3r))
        x2_ref[...] = x2

        @pl.when(i == 0)
        def _():
            aggx_ref[...] = jnp.zeros((NG, H), F32)
            if with_xw:
                aggxw_ref[...] = jnp.zeros((NG, H), F32)

        aggx_ref[...] += _dot(ohT, x2)
        if with_xw:
            aggxw_ref[...] += _dot(ohT, x2 * xpcat)

    ins = list(x_parts) + list(aggn_parts) + [b_a, b_b] + list(u_parts) + \
        [w1xa, w1u, b1, l2[0], l2[1], l3[0], l3[1]]
    specs = [pl.BlockSpec((BLK, p.shape[1]), lambda i: (i, 0))
             for p in list(x_parts) + list(aggn_parts)]
    specs += [pl.BlockSpec((1, BLK, 1), lambda i: (i, 0, 0)),
              pl.BlockSpec((1, 1, BLK), lambda i: (i, 0, 0))]
    specs += [_wspec(a) for a in ins[nx + na + 2:]]

    out_shape = [SDS((R, H), F32), SDS((NG, H), F32)]
    out_specs = [pl.BlockSpec((BLK, H), lambda i: (i, 0)),
                 pl.BlockSpec((NG, H), lambda i: (0, 0))]
    if with_xw:
        out_shape.append(SDS((NG, H), F32))
        out_specs.append(pl.BlockSpec((NG, H), lambda i: (0, 0)))
    return pl.pallas_call(
        body, grid=(R // BLK,), in_specs=specs, out_specs=tuple(out_specs),
        out_shape=tuple(out_shape))(*ins)


def _useg(aggx, aggg, u_parts, layers):
    nu = len(u_parts)
    (w1, b1), l2, l3 = layers

    def body(*refs):
        it = iter(refs)
        ax = next(it); ag = next(it)
        urs = [next(it) for _ in range(nu)]
        w1r = next(it); b1r = next(it)
        w2r = next(it); b2r = next(it); w3r = next(it); b3r = next(it)
        o_ref = next(it)
        xin = jnp.concatenate([ax[...], ag[...]] + [r[...] for r in urs], axis=1)
        h = jnp.maximum(_dot(xin, w1r[...]) + b1r[...], 0.0)
        o_ref[...] = _mlp23(h, (w2r, b2r), (w3r, b3r))

    ins = [aggx, aggg] + list(u_parts) + [w1, b1, l2[0], l2[1], l3[0], l3[1]]
    specs = [_wspec(a) for a in ins]
    return pl.pallas_call(
        body, grid=(1,), in_specs=specs,
        out_specs=pl.BlockSpec((NG, H), lambda i: (0, 0)),
        out_shape=SDS((NG, H), F32))(*ins)


def _final(aggxw, aggew, u_a, u_h, lay_agg, lay_fin):
    def body(axw, aew, uar, uhr,
             aw1, ab1, aw2, ab2, aw3, ab3,
             fw1, fb1, fw2, fb2, fw3, fb3, o_ref):
        uw = uar[...] * uhr[...]
        gin = jnp.concatenate([axw[...], aew[...], uw], axis=1)
        h = jnp.maximum(_dot(gin, aw1[...]) + ab1[...], 0.0)
        g = _mlp23(h, (aw2, ab2), (aw3, ab3))
        d = g[0:8, :] - g[8:16, :]
        h2 = jnp.maximum(_dot(d, fw1[...]) + fb1[...], 0.0)
        o_ref[...] = _mlp23(h2, (fw2, fb2), (fw3, fb3))

    (aw1, ab1), (aw2, ab2), (aw3, ab3) = lay_agg
    (fw1, fb1), (fw2, fb2), (fw3, fb3) = lay_fin
    ins = [aggxw, aggew, u_a, u_h,
           aw1, ab1, aw2, ab2, aw3, ab3, fw1, fb1, fw2, fb2, fw3, fb3]
    specs = [_wspec(a) for a in ins]
    return pl.pallas_call(
        body, grid=(1,), in_specs=specs,
        out_specs=pl.BlockSpec((8, 64), lambda i: (0, 0)),
        out_shape=SDS((8, 64), F32))(*ins)


_MESH = dict(core_axis_name="c", subcore_axis_name="s")
_NC, _NS = 2, 16
_NW = _NC * _NS


def _sc_gather_diff(a, dst, src):
    E = dst.shape[0]
    CH = 80
    per_w = E // _NW
    nch = per_w // CH
    npairs = nch // 2

    @functools.partial(
        pl.kernel,
        out_type=SDS((E, H), F32),
        mesh=plsc.VectorSubcoreMesh(**_MESH),
        scratch_types=[pltpu.VMEM((per_w,), I32), pltpu.VMEM((per_w,), I32),
                       pltpu.VMEM((CH, H), F32), pltpu.VMEM((CH, H), F32),
                       pltpu.VMEM((CH, H), F32),
                       pltpu.VMEM((CH, H), F32), pltpu.VMEM((CH, H), F32),
                       pltpu.VMEM((CH, H), F32),
                       pltpu.SemaphoreType.DMA, pltpu.SemaphoreType.DMA,
                       pltpu.SemaphoreType.DMA, pltpu.SemaphoreType.DMA],
    )
    def k(a_hbm, dst_hbm, src_hbm, d_hbm,
          idx_d, idx_s, r1a, r2a, wa, r1b, r2b, wb, gsa, gsb, wsa, wsb):
        cid = lax.axis_index("c")
        sid = lax.axis_index("s")
        wid = sid * _NC + cid
        ebase = wid * per_w
        pltpu.sync_copy(dst_hbm.at[pl.ds(ebase, per_w)], idx_d)
        pltpu.sync_copy(src_hbm.at[pl.ds(ebase, per_w)], idx_s)

        def fire(ch, r1, r2, gs):
            off = ch * CH
            c1 = pltpu.async_copy(a_hbm.at[idx_d.at[pl.ds(off, CH)]], r1, gs)
            c2 = pltpu.async_copy(a_hbm.at[idx_s.at[pl.ds(off, CH)]], r2, gs)
            return c1, c2

        def sub(r1, r2, w):
            def row(r, carry):
                for j in range(H // 16):
                    s = pl.ds(j * 16, 16)
                    w[r, s] = r1[r, s] - r2[r, s]
                return carry

            lax.fori_loop(0, CH, row, 0)

        def wwait(w, ws):
            pltpu.make_async_copy(w, d_hbm.at[pl.ds(0, CH)], ws).wait()

        def proc(k_, ch, descs, r1, r2, w, ws):
            descs[0].wait()
            descs[1].wait()

            @pl.when(k_ > 0)
            def _():
                wwait(w, ws)

            sub(r1, r2, w)
            pltpu.async_copy(w, d_hbm.at[pl.ds(ebase + ch * CH, CH)], ws)

        def body(t, carry):
            ca = 2 * t
            cb = 2 * t + 1
            da = fire(ca, r1a, r2a, gsa)
            db = fire(cb, r1b, r2b, gsb)
            proc(t, ca, da, r1a, r2a, wa, wsa)
            proc(t, cb, db, r1b, r2b, wb, wsb)
            return carry

        lax.fori_loop(0, npairs, body, 0)
        dl = fire(nch - 1, r1a, r2a, gsa)
        proc(jnp.int32(1), nch - 1, dl, r1a, r2a, wa, wsa)
        wwait(wa, wsa)
        wwait(wb, wsb)

    return k(a, dst, src)


def _sc_scatter(e2, dst3):
    CH = 80
    nch = dst3.shape[1]
    npairs = nch // 2
    per_w = nch * CH
    ZR = 16
    rsmall, rbig = 624, 640

    @functools.partial(
        pl.kernel,
        out_type=SDS((2 * N1, H), F32),
        mesh=plsc.VectorSubcoreMesh(**_MESH),
        scratch_types=[pltpu.VMEM((2, 1, CH), I32),
                       pltpu.VMEM((CH, H), F32), pltpu.VMEM((CH, H), F32),
                       pltpu.VMEM((ZR, H), F32),
                       pltpu.VMEM_SHARED((N1, H), F32),
                       pltpu.SemaphoreType.DMA, pltpu.SemaphoreType.DMA],
    )
    def k(e2_hbm, dst_hbm, out_hbm, ibuf, ra, rb, zbuf, acc, sa, sb):
        cid = lax.axis_index("c")
        sid = lax.axis_index("s")
        wid = cid * _NS + sid
        nbase = cid * N1

        def zrow(r, carry):
            for j in range(H // 16):
                zbuf[r, pl.ds(j * 16, 16)] = jnp.zeros((16,), F32)
            return carry

        lax.fori_loop(0, ZR, zrow, 0)
        rbase = sid * rsmall
        last = _NS - 1

        def zcopy(t, carry):
            pltpu.sync_copy(zbuf, acc.at[pl.ds(rbase + t * ZR, ZR)])
            return carry

        lax.fori_loop(0, rsmall // ZR, zcopy, 0)

        @pl.when(sid == last)
        def _():
            pltpu.sync_copy(zbuf, acc.at[pl.ds(last * rsmall + rsmall, ZR)])

        plsc.subcore_barrier()

        def stage_idx(ch, slot):
            pltpu.sync_copy(dst_hbm.at[wid].at[ch], ibuf.at[slot])
            for j in range(CH // 16):
                s = pl.ds(j * 16, 16)
                ibuf[slot, 0, s] = ibuf[slot, 0, s] - nbase

        def fire(ch, rbuf, sem):
            off = wid * per_w + ch * CH
            return pltpu.async_copy(e2_hbm.at[pl.ds(off, CH)], rbuf, sem)

        def body(t, carry):
            ca = 2 * t
            cb = 2 * t + 1
            da = fire(ca, ra, sa)
            db = fire(cb, rb, sb)
            stage_idx(ca, 0)
            stage_idx(cb, 1)
            da.wait()
            pltpu.sync_copy(ra, acc.at[ibuf.at[0, 0]], add=True)
            db.wait()
            pltpu.sync_copy(rb, acc.at[ibuf.at[1, 0]], add=True)
            return carry

        lax.fori_loop(0, npairs, body, 0)
        dl = fire(nch - 1, ra, sa)
        stage_idx(nch - 1, 0)
        dl.wait()
        pltpu.sync_copy(ra, acc.at[ibuf.at[0, 0]], add=True)
        plsc.subcore_barrier()

        pltpu.sync_copy(acc.at[pl.ds(rbase, rsmall)],
                        out_hbm.at[pl.ds(nbase + rbase, rsmall)])

        @pl.when(sid == last)
        def _():
            pltpu.sync_copy(acc.at[pl.ds(last * rsmall + rsmall, ZR)],
                            out_hbm.at[pl.ds(nbase + last * rsmall + rsmall, ZR)])

    return k(e2, dst3)


def _meta(pe, px, pu, x_parts, e_parts, u_parts, gidx, attention=False):
    dst, dst3, src, src_a, src_b, cum, b_a, b_b = gidx
    npart = len(x_parts)
    K = npart * H

    (we1, be1), el2, el3 = _full_w(pe)
    (wx1, bx1), xl2, xl3 = _full_w(px)

    w1x = we1[0:K]
    w1e = we1[K:2 * K]
    w1u = we1[2 * K:3 * K]
    a = _a_proj(x_parts, w1x)
    d = _sc_gather_diff(a, dst, src)

    eouts = _edge(d, e_parts, src_a, src_b, cum, u_parts,
                  w1e, w1u, be1, el2, el3, with_ew=attention)
    if attention:
        e2, aggg, aggew = eouts
    else:
        e2, aggg = eouts
        aggew = None

    aggn = _sc_scatter(e2, dst3)

    w1xa = wx1[0:K + H]
    w1xu = wx1[K + H:K + H + npart * H]
    nouts = _node(x_parts, [aggn], b_a, b_b, u_parts,
                  w1xa, w1xu, bx1, xl2, xl3, with_xw=attention)
    if attention:
        x2, aggx, aggxw = nouts
    else:
        x2, aggx = nouts
        aggxw = None

    u2 = _useg(aggx, aggg, u_parts, _full_w(pu))
    return x2, e2, u2, (aggxw, aggew)


def kernel(x1, edge_index1, e1, u1, batch1, x2, edge_index2, e2, u2, batch2, params):
    x = jnp.concatenate([x1, x2], axis=0)
    e = jnp.concatenate([e1, e2], axis=0)
    u = jnp.concatenate([u1, u2], axis=0)
    batch = jnp.concatenate(
        [batch1.astype(I32), batch2.astype(I32) + 8], axis=0)
    src = jnp.concatenate(
        [edge_index1[0].astype(I32), edge_index2[0].astype(I32) + N1])
    dst = jnp.concatenate(
        [edge_index1[1].astype(I32), edge_index2[1].astype(I32) + N1])

    E = src.shape[0]
    N = x.shape[0]

    src_a = src.reshape(E // BLK, BLK, 1)
    src_b = src.reshape(E // BLK, 1, BLK)
    b_a = batch.reshape(N // BLK, BLK, 1)
    b_b = batch.reshape(N // BLK, 1, BLK)
    dst3 = dst.reshape(_NW, E // (_NW * 80), 1, 80)
    cum = _graph_starts(b_a, b_b)
    gidx = (dst, dst3, src, src_a, src_b, cum, b_a, b_b)

    p = params
    x_h = _enc(x, _full_w(p['enc_x']))
    e_h = _enc(e, _full_w(p['enc_e']))
    u_h = _enc(u, _full_w(p['enc_u']))

    for _ in range(3):
        x_h, e_h, u_h, _unused = _meta(
            p['rec_e'], p['rec_x'], p['rec_u'],
            [x, x_h], [e, e_h], [u, u_h], gidx, attention=False)

    _xa, _ea, u_a, (aggxw, aggew) = _meta(
        p['att_e'], p['att_x'], p['att_u'],
        [x_h], [e_h], [u_h], gidx, attention=True)

    return _final(aggxw, aggew, u_a, u_h,
                  _full_w(p['agg_u']), _full_w(p['final']))


# speedup vs baseline: 4.6415x; 1.2611x over previous
import functools

import jax
import jax.numpy as jnp
from jax import lax
from jax.experimental import pallas as pl
from jax.experimental.pallas import tpu as pltpu
from jax.experimental.pallas import tpu_sc as plsc

F32 = jnp.float32
I32 = jnp.int32
SDS = jax.ShapeDtypeStruct
PREC = lax.Precision.DEFAULT

N1 = 10000
E1 = 160000
NG = 16
H = 128

BLK = 2000


def _dot(a, b):
    return lax.dot_general(a, b, (((a.ndim - 1,), (0,)), ((), ())),
                           precision=PREC, preferred_element_type=F32)


def _mlp23(h, l2, l3):
    (w2, b2), (w3, b3) = l2, l3
    h = jnp.maximum(_dot(h, w2[...]) + b2[...], 0.0)
    return _dot(h, w3[...]) + b3[...]


def _wspec(arr):
    nd = arr.ndim
    return pl.BlockSpec(arr.shape, lambda i: (0,) * nd)


def _full_w(params):
    return [(w, b.reshape(1, -1)) for (w, b) in params]


def _enc(x, layers):
    R = x.shape[0]
    blk = BLK if R % BLK == 0 else R
    (w1, b1), l2, l3 = layers

    def body(x_ref, w1r, b1r, w2r, b2r, w3r, b3r, o_ref):
        h = jnp.maximum(_dot(x_ref[...], w1r[...]) + b1r[...], 0.0)
        o_ref[...] = _mlp23(h, (w2r, b2r), (w3r, b3r))

    ins = [x, w1, b1, l2[0], l2[1], l3[0], l3[1]]
    specs = [pl.BlockSpec((blk, x.shape[1]), lambda i: (i, 0))] + [_wspec(a) for a in ins[1:]]
    return pl.pallas_call(
        body, grid=(R // blk,), in_specs=specs,
        out_specs=pl.BlockSpec((blk, H), lambda i: (i, 0)),
        out_shape=SDS((R, H), F32))(*ins)


def _a_proj(x_parts, w1x):
    R = x_parts[0].shape[0]
    np_ = len(x_parts)

    def body(*refs):
        xr = refs[:np_]
        wr = refs[np_]
        o_ref = refs[np_ + 1]
        xc = jnp.concatenate([r[...] for r in xr], axis=1)
        o_ref[...] = _dot(xc, wr[...])

    ins = list(x_parts) + [w1x]
    specs = [pl.BlockSpec((BLK, H), lambda i: (i, 0)) for _ in x_parts] + [_wspec(w1x)]
    return pl.pallas_call(
        body, grid=(R // BLK,), in_specs=specs,
        out_specs=pl.BlockSpec((BLK, H), lambda i: (i, 0)),
        out_shape=SDS((R, H), F32))(*ins)


def _graph_starts(b_a, b_b):
    nblk = b_a.shape[0]

    def body(bar, bbr, row_ref, col_ref):
        i = pl.program_id(0)
        ids_a = bar[...][0]
        ids_b = bbr[...][0]
        lt = (ids_a < lax.broadcasted_iota(I32, (BLK, NG), 1)).astype(F32)
        ltT = (lax.broadcasted_iota(I32, (NG, BLK), 0) > ids_b).astype(F32)

        @pl.when(i == 0)
        def _():
            row_ref[...] = jnp.zeros((1, NG), F32)
            col_ref[...] = jnp.zeros((NG, 1), F32)

        row_ref[...] += _dot(jnp.ones((1, BLK), F32), lt)
        col_ref[...] += _dot(ltT, jnp.ones((BLK, 1), F32))

    return pl.pallas_call(
        body, grid=(nblk,),
        in_specs=[pl.BlockSpec((1, BLK, 1), lambda i: (i, 0, 0)),
                  pl.BlockSpec((1, 1, BLK), lambda i: (i, 0, 0))],
        out_specs=(pl.BlockSpec((1, NG), lambda i: (0, 0)),
                   pl.BlockSpec((NG, 1), lambda i: (0, 0))),
        out_shape=(SDS((1, NG), F32), SDS((NG, 1), F32)))(b_a, b_b)


def _seg_onehots(ids_a, ids_b, cum_row, cum_col):
    fa = ids_a.astype(F32)
    fb = ids_b.astype(F32)
    ge = (fa >= cum_row).astype(F32)
    geT = (fb >= cum_col).astype(F32)
    oh = ge - jnp.concatenate([ge[:, 1:], jnp.zeros((ge.shape[0], 1), F32)], 1)
    ohT = geT - jnp.concatenate([geT[1:, :], jnp.zeros((1, geT.shape[1]), F32)], 0)
    return oh, ohT


def _edge(d, e_parts, src_a, src_b, cum, u_parts, w1e, w1u, b1, l2, l3,
          with_ew=False):
    E = d.shape[0]
    cum_row, cum_col = cum
    ne, nu = len(e_parts), len(u_parts)

    def body(*refs):
        it = iter(refs)
        dr = next(it)
        ers = [next(it) for _ in range(ne)]
        sar = next(it); sbr = next(it)
        crr = next(it); ccr = next(it)
        urs = [next(it) for _ in range(nu)]
        w1er = next(it); w1ur = next(it); b1r = next(it)
        w2r = next(it); b2r = next(it); w3r = next(it); b3r = next(it)
        e2_ref = next(it)
        aggg_ref = next(it)
        aggew_ref = next(it) if with_ew else None

        i = pl.program_id(0)
        ecat = jnp.concatenate([r[...] for r in ers], axis=1)
        ucat = jnp.concatenate([r[...] for r in urs], axis=1)
        gu = _dot(ucat, w1ur[...]) + b1r[...]
        oh, ohT = _seg_onehots(sar[...][0], sbr[...][0], crr[...], ccr[...])
        h = dr[...] + _dot(ecat, w1er[...]) + _dot(oh, gu)
        h = jnp.maximum(h, 0.0)
        e2 = _mlp23(h, (w2r, b2r), (w3r, b3r))
        e2_ref[...] = e2

        @pl.when(i == 0)
        def _():
            aggg_ref[...] = jnp.zeros((NG, H), F32)
            if with_ew:
                aggew_ref[...] = jnp.zeros((NG, H), F32)

        aggg_ref[...] += _dot(ohT, e2)
        if with_ew:
            aggew_ref[...] += _dot(ohT, e2 * ecat)

    ins = [d] + list(e_parts) + [src_a, src_b, cum_row, cum_col] + \
        list(u_parts) + [w1e, w1u, b1, l2[0], l2[1], l3[0], l3[1]]
    specs = [pl.BlockSpec((BLK, H), lambda i: (i, 0))]
    specs += [pl.BlockSpec((BLK, p.shape[1]), lambda i: (i, 0)) for p in e_parts]
    specs += [pl.BlockSpec((1, BLK, 1), lambda i: (i, 0, 0)),
              pl.BlockSpec((1, 1, BLK), lambda i: (i, 0, 0))]
    specs += [_wspec(a) for a in ins[3 + ne:]]

    out_shape = [SDS((E, H), F32), SDS((NG, H), F32)]
    out_specs = [pl.BlockSpec((BLK, H), lambda i: (i, 0)),
                 pl.BlockSpec((NG, H), lambda i: (0, 0))]
    if with_ew:
        out_shape.append(SDS((NG, H), F32))
        out_specs.append(pl.BlockSpec((NG, H), lambda i: (0, 0)))
    return pl.pallas_call(
        body, grid=(E // BLK,), in_specs=specs, out_specs=tuple(out_specs),
        out_shape=tuple(out_shape))(*ins)


def _node(x_parts, aggn_parts, b_a, b_b, u_parts, w1xa, w1u, b1, l2, l3,
          with_xw=False):
    R = x_parts[0].shape[0]
    nx, na, nu = len(x_parts), len(aggn_parts), len(u_parts)

    def body(*refs):
        it = iter(refs)
        xrs = [next(it) for _ in range(nx)]
        ars = [next(it) for _ in range(na)]
        bar = next(it); bbr = next(it)
        urs = [next(it) for _ in range(nu)]
        w1r = next(it); w1ur = next(it); b1r = next(it)
        w2r = next(it); b2r = next(it); w3r = next(it); b3r = next(it)
        x2_ref = next(it)
        aggx_ref = next(it)
        aggxw_ref = next(it) if with_xw else None

        i = pl.program_id(0)
        xpcat = jnp.concatenate([r[...] for r in xrs], axis=1)
        xc = jnp.concatenate([xpcat] + [r[...] for r in ars], axis=1)
        ucat = jnp.concatenate([r[...] for r in urs], axis=1)
        gu = _dot(ucat, w1ur[...]) + b1r[...]
        ids_a = bar[...][0]
        ids_b = bbr[...][0]
        oh = (ids_a == lax.broadcasted_iota(I32, (BLK, NG), 1)).astype(F32)
        ohT = (ids_b == lax.broadcasted_iota(I32, (NG, BLK), 0)).astype(F32)
        h = jnp.maximum(_dot(xc, w1r[...]) + _dot(oh, gu), 0.0)
        x2 = _mlp23(h, (w2r, b2r), (w3r, b3r))
        x2_ref[...] = x2

        @pl.when(i == 0)
        def _():
            aggx_ref[...] = jnp.zeros((NG, H), F32)
            if with_xw:
                aggxw_ref[...] = jnp.zeros((NG, H), F32)

        aggx_ref[...] += _dot(ohT, x2)
        if with_xw:
            aggxw_ref[...] += _dot(ohT, x2 * xpcat)

    ins = list(x_parts) + list(aggn_parts) + [b_a, b_b] + list(u_parts) + \
        [w1xa, w1u, b1, l2[0], l2[1], l3[0], l3[1]]
    specs = [pl.BlockSpec((BLK, p.shape[1]), lambda i: (i, 0))
             for p in list(x_parts) + list(aggn_parts)]
    specs += [pl.BlockSpec((1, BLK, 1), lambda i: (i, 0, 0)),
              pl.BlockSpec((1, 1, BLK), lambda i: (i, 0, 0))]
    specs += [_wspec(a) for a in ins[nx + na + 2:]]

    out_shape = [SDS((R, H), F32), SDS((NG, H), F32)]
    out_specs = [pl.BlockSpec((BLK, H), lambda i: (i, 0)),
                 pl.BlockSpec((NG, H), lambda i: (0, 0))]
    if with_xw:
        out_shape.append(SDS((NG, H), F32))
        out_specs.append(pl.BlockSpec((NG, H), lambda i: (0, 0)))
    return pl.pallas_call(
        body, grid=(R // BLK,), in_specs=specs, out_specs=tuple(out_specs),
        out_shape=tuple(out_shape))(*ins)


def _useg(aggx, aggg, u_parts, layers):
    nu = len(u_parts)
    (w1, b1), l2, l3 = layers

    def body(*refs):
        it = iter(refs)
        ax = next(it); ag = next(it)
        urs = [next(it) for _ in range(nu)]
        w1r = next(it); b1r = next(it)
        w2r = next(it); b2r = next(it); w3r = next(it); b3r = next(it)
        o_ref = next(it)
        xin = jnp.concatenate([ax[...], ag[...]] + [r[...] for r in urs], axis=1)
        h = jnp.maximum(_dot(xin, w1r[...]) + b1r[...], 0.0)
        o_ref[...] = _mlp23(h, (w2r, b2r), (w3r, b3r))

    ins = [aggx, aggg] + list(u_parts) + [w1, b1, l2[0], l2[1], l3[0], l3[1]]
    specs = [_wspec(a) for a in ins]
    return pl.pallas_call(
        body, grid=(1,), in_specs=specs,
        out_specs=pl.BlockSpec((NG, H), lambda i: (0, 0)),
        out_shape=SDS((NG, H), F32))(*ins)


def _final(aggxw, aggew, u_a, u_h, lay_agg, lay_fin):
    def body(axw, aew, uar, uhr,
             aw1, ab1, aw2, ab2, aw3, ab3,
             fw1, fb1, fw2, fb2, fw3, fb3, o_ref):
        uw = uar[...] * uhr[...]
        gin = jnp.concatenate([axw[...], aew[...], uw], axis=1)
        h = jnp.maximum(_dot(gin, aw1[...]) + ab1[...], 0.0)
        g = _mlp23(h, (aw2, ab2), (aw3, ab3))
        d = g[0:8, :] - g[8:16, :]
        h2 = jnp.maximum(_dot(d, fw1[...]) + fb1[...], 0.0)
        o_ref[...] = _mlp23(h2, (fw2, fb2), (fw3, fb3))

    (aw1, ab1), (aw2, ab2), (aw3, ab3) = lay_agg
    (fw1, fb1), (fw2, fb2), (fw3, fb3) = lay_fin
    ins = [aggxw, aggew, u_a, u_h,
           aw1, ab1, aw2, ab2, aw3, ab3, fw1, fb1, fw2, fb2, fw3, fb3]
    specs = [_wspec(a) for a in ins]
    return pl.pallas_call(
        body, grid=(1,), in_specs=specs,
        out_specs=pl.BlockSpec((8, 64), lambda i: (0, 0)),
        out_shape=SDS((8, 64), F32))(*ins)


_MESH = dict(core_axis_name="c", subcore_axis_name="s")
_NC, _NS = 2, 16
_NW = _NC * _NS


def _sc_gather_diff(a, dst, src):
    E = dst.shape[0]
    CH = 80
    per_w = E // _NW
    nch = per_w // CH
    npairs = nch // 2

    @functools.partial(
        pl.kernel,
        out_type=SDS((E, H), F32),
        mesh=plsc.VectorSubcoreMesh(**_MESH),
        scratch_types=[pltpu.VMEM((per_w,), I32), pltpu.VMEM((per_w,), I32),
                       pltpu.VMEM((CH, H), F32), pltpu.VMEM((CH, H), F32),
                       pltpu.VMEM((CH, H), F32),
                       pltpu.VMEM((CH, H), F32), pltpu.VMEM((CH, H), F32),
                       pltpu.VMEM((CH, H), F32),
                       pltpu.SemaphoreType.DMA, pltpu.SemaphoreType.DMA,
                       pltpu.SemaphoreType.DMA, pltpu.SemaphoreType.DMA],
    )
    def k(a_hbm, dst_hbm, src_hbm, d_hbm,
          idx_d, idx_s, r1a, r2a, wa, r1b, r2b, wb, gsa, gsb, wsa, wsb):
        cid = lax.axis_index("c")
        sid = lax.axis_index("s")
        wid = sid * _NC + cid
        ebase = wid * per_w
        pltpu.sync_copy(dst_hbm.at[pl.ds(ebase, per_w)], idx_d)
        pltpu.sync_copy(src_hbm.at[pl.ds(ebase, per_w)], idx_s)

        def fire(ch, r1, r2, gs):
            off = ch * CH
            c1 = pltpu.async_copy(a_hbm.at[idx_d.at[pl.ds(off, CH)]], r1, gs)
            c2 = pltpu.async_copy(a_hbm.at[idx_s.at[pl.ds(off, CH)]], r2, gs)
            return c1, c2

        def sub(r1, r2, w):
            def row(r, carry):
                for j in range(H // 16):
                    s = pl.ds(j * 16, 16)
                    w[r, s] = r1[r, s] - r2[r, s]
                return carry

            lax.fori_loop(0, CH, row, 0)

        def wwait(w, ws):
            pltpu.make_async_copy(w, d_hbm.at[pl.ds(0, CH)], ws).wait()

        def proc(k_, ch, descs, r1, r2, w, ws):
            descs[0].wait()
            descs[1].wait()

            @pl.when(k_ > 0)
            def _():
                wwait(w, ws)

            sub(r1, r2, w)
            pltpu.async_copy(w, d_hbm.at[pl.ds(ebase + ch * CH, CH)], ws)

        def body(t, carry):
            ca = 2 * t
            cb = 2 * t + 1
            da = fire(ca, r1a, r2a, gsa)
            db = fire(cb, r1b, r2b, gsb)
            proc(t, ca, da, r1a, r2a, wa, wsa)
            proc(t, cb, db, r1b, r2b, wb, wsb)
            return carry

        lax.fori_loop(0, npairs, body, 0)
        dl = fire(nch - 1, r1a, r2a, gsa)
        proc(jnp.int32(1), nch - 1, dl, r1a, r2a, wa, wsa)
        wwait(wa, wsa)
        wwait(wb, wsb)

    return k(a, dst, src)


def _sc_scatter(e2, dst3):
    CH = 80
    nch = dst3.shape[1]
    npairs = nch // 2
    per_w = nch * CH
    ZR = 16
    rsmall, rbig = 624, 640

    @functools.partial(
        pl.kernel,
        out_type=SDS((2 * N1, H), F32),
        mesh=plsc.VectorSubcoreMesh(**_MESH),
        scratch_types=[pltpu.VMEM((2, 1, CH), I32),
                       pltpu.VMEM((CH, H), F32), pltpu.VMEM((CH, H), F32),
                       pltpu.VMEM((ZR, H), F32),
                       pltpu.VMEM_SHARED((N1, H), F32),
                       pltpu.SemaphoreType.DMA, pltpu.SemaphoreType.DMA],
    )
    def k(e2_hbm, dst_hbm, out_hbm, ibuf, ra, rb, zbuf, acc, sa, sb):
        cid = lax.axis_index("c")
        sid = lax.axis_index("s")
        wid = cid * _NS + sid
        nbase = cid * N1

        def zrow(r, carry):
            for j in range(H // 16):
                zbuf[r, pl.ds(j * 16, 16)] = jnp.zeros((16,), F32)
            return carry

        lax.fori_loop(0, ZR, zrow, 0)
        rbase = sid * rsmall
        last = _NS - 1

        def zcopy(t, carry):
            pltpu.sync_copy(zbuf, acc.at[pl.ds(rbase + t * ZR, ZR)])
            return carry

        lax.fori_loop(0, rsmall // ZR, zcopy, 0)

        @pl.when(sid == last)
        def _():
            pltpu.sync_copy(zbuf, acc.at[pl.ds(last * rsmall + rsmall, ZR)])

        plsc.subcore_barrier()

        def stage_idx(ch, slot):
            pltpu.sync_copy(dst_hbm.at[wid].at[ch], ibuf.at[slot])
            for j in range(CH // 16):
                s = pl.ds(j * 16, 16)
                ibuf[slot, 0, s] = ibuf[slot, 0, s] - nbase

        def fire(ch, rbuf, sem):
            off = wid * per_w + ch * CH
            return pltpu.async_copy(e2_hbm.at[pl.ds(off, CH)], rbuf, sem)

        def body(t, carry):
            ca = 2 * t
            cb = 2 * t + 1
            da = fire(ca, ra, sa)
            db = fire(cb, rb, sb)
            stage_idx(ca, 0)
            stage_idx(cb, 1)
            da.wait()
            pltpu.sync_copy(ra, acc.at[ibuf.at[0, 0]], add=True)
            db.wait()
            pltpu.sync_copy(rb, acc.at[ibuf.at[1, 0]], add=True)
            return carry

        lax.fori_loop(0, npairs, body, 0)
        dl = fire(nch - 1, ra, sa)
        stage_idx(nch - 1, 0)
        dl.wait()
        pltpu.sync_copy(ra, acc.at[ibuf.at[0, 0]], add=True)
        plsc.subcore_barrier()

        pltpu.sync_copy(acc.at[pl.ds(rbase, rsmall)],
                        out_hbm.at[pl.ds(nbase + rbase, rsmall)])

        @pl.when(sid == last)
        def _():
            pltpu.sync_copy(acc.at[pl.ds(last * rsmall + rsmall, ZR)],
                            out_hbm.at[pl.ds(nbase + last * rsmall + rsmall, ZR)])

    return k(e2, dst3)


def _meta(pe, px, pu, x_parts, e_parts, u_parts, gidx, attention=False):
    dst, dst3, src, src_a, src_b, cum, b_a, b_b = gidx
    npart = len(x_parts)
    K = npart * H

    (we1, be1), el2, el3 = _full_w(pe)
    (wx1, bx1), xl2, xl3 = _full_w(px)

    w1x = we1[0:K]
    w1e = we1[K:2 * K]
    w1u = we1[2 * K:3 * K]
    a = _a_proj(x_parts, w1x)
    d = _sc_gather_diff(a, dst, src)

    eouts = _edge(d, e_parts, src_a, src_b, cum, u_parts,
                  w1e, w1u, be1, el2, el3, with_ew=attention)
    if attention:
        e2, aggg, aggew = eouts
    else:
        e2, aggg = eouts
        aggew = None

    aggn = _sc_scatter(e2, dst3)

    w1xa = wx1[0:K + H]
    w1xu = wx1[K + H:K + H + npart * H]
    nouts = _node(x_parts, [aggn], b_a, b_b, u_parts,
                  w1xa, w1xu, bx1, xl2, xl3, with_xw=attention)
    if attention:
        x2, aggx, aggxw = nouts
    else:
        x2, aggx = nouts
        aggxw = None

    u2 = _useg(aggx, aggg, u_parts, _full_w(pu))
    return x2, e2, u2, (aggxw, aggew)


def kernel(x1, edge_index1, e1, u1, batch1, x2, edge_index2, e2, u2, batch2, params):
    x = jnp.concatenate([x1, x2], axis=0)
    e = jnp.concatenate([e1, e2], axis=0)
    u = jnp.concatenate([u1, u2], axis=0)
    batch = jnp.concatenate(
        [batch1.astype(I32), batch2.astype(I32) + 8], axis=0)
    src = jnp.concatenate(
        [edge_index1[0].astype(I32), edge_index2[0].astype(I32) + N1])
    dst = jnp.concatenate(
        [edge_index1[1].astype(I32), edge_index2[1].astype(I32) + N1])

    E = src.shape[0]
    N = x.shape[0]

    src_a = src.reshape(E // BLK, BLK, 1)
    src_b = src.reshape(E // BLK, 1, BLK)
    b_a = batch.reshape(N // BLK, BLK, 1)
    b_b = batch.reshape(N // BLK, 1, BLK)
    dst3 = dst.reshape(_NW, E // (_NW * 80), 1, 80)
    cum = _graph_starts(b_a, b_b)
    gidx = (dst, dst3, src, src_a, src_b, cum, b_a, b_b)

    p = params
    x_h = _enc(x, _full_w(p['enc_x']))
    e_h = _enc(e, _full_w(p['enc_e']))
    u_h = _enc(u, _full_w(p['enc_u']))

    for _ in range(3):
        x_h, e_h, u_h, _unused = _meta(
            p['rec_e'], p['rec_x'], p['rec_u'],
            [x, x_h], [e, e_h], [u, u_h], gidx, attention=False)

    _xa, _ea, u_a, (aggxw, aggew) = _meta(
        p['att_e'], p['att_x'], p['att_u'],
        [x_h], [e_h], [u_h], gidx, attention=True)

    return _final(aggxw, aggew, u_a, u_h,
                  _full_w(p['agg_u']), _full_w(p['final']))


# speedup vs baseline: 5.1065x; 1.1002x over previous
import functools

import jax
import jax.numpy as jnp
from jax import lax
from jax.experimental import pallas as pl
from jax.experimental.pallas import tpu as pltpu
from jax.experimental.pallas import tpu_sc as plsc

F32 = jnp.float32
I32 = jnp.int32
SDS = jax.ShapeDtypeStruct
PREC = lax.Precision.DEFAULT

N1 = 10000
E1 = 160000
NG = 16
H = 128

BLK = 4000


def _dot(a, b):
    return lax.dot_general(a, b, (((a.ndim - 1,), (0,)), ((), ())),
                           precision=PREC, preferred_element_type=F32)


def _mlp23(h, l2, l3):
    (w2, b2), (w3, b3) = l2, l3
    h = jnp.maximum(_dot(h, w2[...]) + b2[...], 0.0)
    return _dot(h, w3[...]) + b3[...]


def _wspec(arr):
    nd = arr.ndim
    return pl.BlockSpec(arr.shape, lambda i: (0,) * nd)


def _full_w(params):
    return [(w, b.reshape(1, -1)) for (w, b) in params]


def _enc(x, layers):
    R = x.shape[0]
    blk = BLK if R % BLK == 0 else R
    (w1, b1), l2, l3 = layers

    def body(x_ref, w1r, b1r, w2r, b2r, w3r, b3r, o_ref):
        h = jnp.maximum(_dot(x_ref[...], w1r[...]) + b1r[...], 0.0)
        o_ref[...] = _mlp23(h, (w2r, b2r), (w3r, b3r))

    ins = [x, w1, b1, l2[0], l2[1], l3[0], l3[1]]
    specs = [pl.BlockSpec((blk, x.shape[1]), lambda i: (i, 0))] + [_wspec(a) for a in ins[1:]]
    return pl.pallas_call(
        body, grid=(R // blk,), in_specs=specs,
        out_specs=pl.BlockSpec((blk, H), lambda i: (i, 0)),
        out_shape=SDS((R, H), F32))(*ins)


def _a_proj(x_parts, w1x):
    R = x_parts[0].shape[0]
    np_ = len(x_parts)

    def body(*refs):
        xr = refs[:np_]
        wr = refs[np_]
        o_ref = refs[np_ + 1]
        xc = jnp.concatenate([r[...] for r in xr], axis=1)
        o_ref[...] = _dot(xc, wr[...])

    ins = list(x_parts) + [w1x]
    specs = [pl.BlockSpec((BLK, H), lambda i: (i, 0)) for _ in x_parts] + [_wspec(w1x)]
    return pl.pallas_call(
        body, grid=(R // BLK,), in_specs=specs,
        out_specs=pl.BlockSpec((BLK, H), lambda i: (i, 0)),
        out_shape=SDS((R, H), F32))(*ins)


def _graph_starts(b_a, b_b):
    nblk = b_a.shape[0]

    def body(bar, bbr, row_ref, col_ref):
        i = pl.program_id(0)
        ids_a = bar[...][0]
        ids_b = bbr[...][0]
        lt = (ids_a < lax.broadcasted_iota(I32, (BLK, NG), 1)).astype(F32)
        ltT = (lax.broadcasted_iota(I32, (NG, BLK), 0) > ids_b).astype(F32)

        @pl.when(i == 0)
        def _():
            row_ref[...] = jnp.zeros((1, NG), F32)
            col_ref[...] = jnp.zeros((NG, 1), F32)

        row_ref[...] += _dot(jnp.ones((1, BLK), F32), lt)
        col_ref[...] += _dot(ltT, jnp.ones((BLK, 1), F32))

    return pl.pallas_call(
        body, grid=(nblk,),
        in_specs=[pl.BlockSpec((1, BLK, 1), lambda i: (i, 0, 0)),
                  pl.BlockSpec((1, 1, BLK), lambda i: (i, 0, 0))],
        out_specs=(pl.BlockSpec((1, NG), lambda i: (0, 0)),
                   pl.BlockSpec((NG, 1), lambda i: (0, 0))),
        out_shape=(SDS((1, NG), F32), SDS((NG, 1), F32)))(b_a, b_b)


def _seg_onehots(ids_a, ids_b, cum_row, cum_col):
    fa = ids_a.astype(F32)
    fb = ids_b.astype(F32)
    ge = (fa >= cum_row).astype(F32)
    geT = (fb >= cum_col).astype(F32)
    oh = ge - jnp.concatenate([ge[:, 1:], jnp.zeros((ge.shape[0], 1), F32)], 1)
    ohT = geT - jnp.concatenate([geT[1:, :], jnp.zeros((1, geT.shape[1]), F32)], 0)
    return oh, ohT


def _edge(d, e_parts, src_a, src_b, cum, u_parts, w1e, w1u, b1, l2, l3,
          with_ew=False):
    E = d.shape[0]
    cum_row, cum_col = cum
    ne, nu = len(e_parts), len(u_parts)

    def body(*refs):
        it = iter(refs)
        dr = next(it)
        ers = [next(it) for _ in range(ne)]
        sar = next(it); sbr = next(it)
        crr = next(it); ccr = next(it)
        urs = [next(it) for _ in range(nu)]
        w1er = next(it); w1ur = next(it); b1r = next(it)
        w2r = next(it); b2r = next(it); w3r = next(it); b3r = next(it)
        e2_ref = next(it)
        aggg_ref = next(it)
        aggew_ref = next(it) if with_ew else None

        i = pl.program_id(0)
        ecat = jnp.concatenate([r[...] for r in ers], axis=1)
        ucat = jnp.concatenate([r[...] for r in urs], axis=1)
        gu = _dot(ucat, w1ur[...]) + b1r[...]
        oh, ohT = _seg_onehots(sar[...][0], sbr[...][0], crr[...], ccr[...])
        h = dr[...] + _dot(ecat, w1er[...]) + _dot(oh, gu)
        h = jnp.maximum(h, 0.0)
        e2 = _mlp23(h, (w2r, b2r), (w3r, b3r))
        e2_ref[...] = e2

        @pl.when(i == 0)
        def _():
            aggg_ref[...] = jnp.zeros((NG, H), F32)
            if with_ew:
                aggew_ref[...] = jnp.zeros((NG, H), F32)

        aggg_ref[...] += _dot(ohT, e2)
        if with_ew:
            aggew_ref[...] += _dot(ohT, e2 * ecat)

    ins = [d] + list(e_parts) + [src_a, src_b, cum_row, cum_col] + \
        list(u_parts) + [w1e, w1u, b1, l2[0], l2[1], l3[0], l3[1]]
    specs = [pl.BlockSpec((BLK, H), lambda i: (i, 0))]
    specs += [pl.BlockSpec((BLK, p.shape[1]), lambda i: (i, 0)) for p in e_parts]
    specs += [pl.BlockSpec((1, BLK, 1), lambda i: (i, 0, 0)),
              pl.BlockSpec((1, 1, BLK), lambda i: (i, 0, 0))]
    specs += [_wspec(a) for a in ins[3 + ne:]]

    out_shape = [SDS((E, H), F32), SDS((NG, H), F32)]
    out_specs = [pl.BlockSpec((BLK, H), lambda i: (i, 0)),
                 pl.BlockSpec((NG, H), lambda i: (0, 0))]
    if with_ew:
        out_shape.append(SDS((NG, H), F32))
        out_specs.append(pl.BlockSpec((NG, H), lambda i: (0, 0)))
    return pl.pallas_call(
        body, grid=(E // BLK,), in_specs=specs, out_specs=tuple(out_specs),
        out_shape=tuple(out_shape))(*ins)


def _node(x_parts, aggn_parts, b_a, b_b, u_parts, w1xa, w1u, b1, l2, l3,
          with_xw=False):
    R = x_parts[0].shape[0]
    nx, na, nu = len(x_parts), len(aggn_parts), len(u_parts)

    def body(*refs):
        it = iter(refs)
        xrs = [next(it) for _ in range(nx)]
        ars = [next(it) for _ in range(na)]
        bar = next(it); bbr = next(it)
        urs = [next(it) for _ in range(nu)]
        w1r = next(it); w1ur = next(it); b1r = next(it)
        w2r = next(it); b2r = next(it); w3r = next(it); b3r = next(it)
        x2_ref = next(it)
        aggx_ref = next(it)
        aggxw_ref = next(it) if with_xw else None

        i = pl.program_id(0)
        xpcat = jnp.concatenate([r[...] for r in xrs], axis=1)
        xc = jnp.concatenate([xpcat] + [r[...] for r in ars], axis=1)
        ucat = jnp.concatenate([r[...] for r in urs], axis=1)
        gu = _dot(ucat, w1ur[...]) + b1r[...]
        ids_a = bar[...][0]
        ids_b = bbr[...][0]
        oh = (ids_a == lax.broadcasted_iota(I32, (BLK, NG), 1)).astype(F32)
        ohT = (ids_b == lax.broadcasted_iota(I32, (NG, BLK), 0)).astype(F32)
        h = jnp.maximum(_dot(xc, w1r[...]) + _dot(oh, gu), 0.0)
        x2 = _mlp23(h, (w2r, b2r), (w3r, b3r))
        x2_ref[...] = x2

        @pl.when(i == 0)
        def _():
            aggx_ref[...] = jnp.zeros((NG, H), F32)
            if with_xw:
                aggxw_ref[...] = jnp.zeros((NG, H), F32)

        aggx_ref[...] += _dot(ohT, x2)
        if with_xw:
            aggxw_ref[...] += _dot(ohT, x2 * xpcat)

    ins = list(x_parts) + list(aggn_parts) + [b_a, b_b] + list(u_parts) + \
        [w1xa, w1u, b1, l2[0], l2[1], l3[0], l3[1]]
    specs = [pl.BlockSpec((BLK, p.shape[1]), lambda i: (i, 0))
             for p in list(x_parts) + list(aggn_parts)]
    specs += [pl.BlockSpec((1, BLK, 1), lambda i: (i, 0, 0)),
              pl.BlockSpec((1, 1, BLK), lambda i: (i, 0, 0))]
    specs += [_wspec(a) for a in ins[nx + na + 2:]]

    out_shape = [SDS((R, H), F32), SDS((NG, H), F32)]
    out_specs = [pl.BlockSpec((BLK, H), lambda i: (i, 0)),
                 pl.BlockSpec((NG, H), lambda i: (0, 0))]
    if with_xw:
        out_shape.append(SDS((NG, H), F32))
        out_specs.append(pl.BlockSpec((NG, H), lambda i: (0, 0)))
    return pl.pallas_call(
        body, grid=(R // BLK,), in_specs=specs, out_specs=tuple(out_specs),
        out_shape=tuple(out_shape))(*ins)


def _useg(aggx, aggg, u_parts, layers):
    nu = len(u_parts)
    (w1, b1), l2, l3 = layers

    def body(*refs):
        it = iter(refs)
        ax = next(it); ag = next(it)
        urs = [next(it) for _ in range(nu)]
        w1r = next(it); b1r = next(it)
        w2r = next(it); b2r = next(it); w3r = next(it); b3r = next(it)
        o_ref = next(it)
        xin = jnp.concatenate([ax[...], ag[...]] + [r[...] for r in urs], axis=1)
        h = jnp.maximum(_dot(xin, w1r[...]) + b1r[...], 0.0)
        o_ref[...] = _mlp23(h, (w2r, b2r), (w3r, b3r))

    ins = [aggx, aggg] + list(u_parts) + [w1, b1, l2[0], l2[1], l3[0], l3[1]]
    specs = [_wspec(a) for a in ins]
    return pl.pallas_call(
        body, grid=(1,), in_specs=specs,
        out_specs=pl.BlockSpec((NG, H), lambda i: (0, 0)),
        out_shape=SDS((NG, H), F32))(*ins)


def _final(aggxw, aggew, u_a, u_h, lay_agg, lay_fin):
    def body(axw, aew, uar, uhr,
             aw1, ab1, aw2, ab2, aw3, ab3,
             fw1, fb1, fw2, fb2, fw3, fb3, o_ref):
        uw = uar[...] * uhr[...]
        gin = jnp.concatenate([axw[...], aew[...], uw], axis=1)
        h = jnp.maximum(_dot(gin, aw1[...]) + ab1[...], 0.0)
        g = _mlp23(h, (aw2, ab2), (aw3, ab3))
        d = g[0:8, :] - g[8:16, :]
        h2 = jnp.maximum(_dot(d, fw1[...]) + fb1[...], 0.0)
        o_ref[...] = _mlp23(h2, (fw2, fb2), (fw3, fb3))

    (aw1, ab1), (aw2, ab2), (aw3, ab3) = lay_agg
    (fw1, fb1), (fw2, fb2), (fw3, fb3) = lay_fin
    ins = [aggxw, aggew, u_a, u_h,
           aw1, ab1, aw2, ab2, aw3, ab3, fw1, fb1, fw2, fb2, fw3, fb3]
    specs = [_wspec(a) for a in ins]
    return pl.pallas_call(
        body, grid=(1,), in_specs=specs,
        out_specs=pl.BlockSpec((8, 64), lambda i: (0, 0)),
        out_shape=SDS((8, 64), F32))(*ins)


_MESH = dict(core_axis_name="c", subcore_axis_name="s")
_NC, _NS = 2, 16
_NW = _NC * _NS


def _sc_gather_diff(a, dst, src):
    E = dst.shape[0]
    CH = 80
    per_w = E // _NW
    nch = per_w // CH
    npairs = nch // 2

    @functools.partial(
        pl.kernel,
        out_type=SDS((E, H), F32),
        mesh=plsc.VectorSubcoreMesh(**_MESH),
        scratch_types=[pltpu.VMEM((per_w,), I32), pltpu.VMEM((per_w,), I32),
                       pltpu.VMEM((CH, H), F32), pltpu.VMEM((CH, H), F32),
                       pltpu.VMEM((CH, H), F32),
                       pltpu.VMEM((CH, H), F32), pltpu.VMEM((CH, H), F32),
                       pltpu.VMEM((CH, H), F32),
                       pltpu.SemaphoreType.DMA, pltpu.SemaphoreType.DMA,
                       pltpu.SemaphoreType.DMA, pltpu.SemaphoreType.DMA],
    )
    def k(a_hbm, dst_hbm, src_hbm, d_hbm,
          idx_d, idx_s, r1a, r2a, wa, r1b, r2b, wb, gsa, gsb, wsa, wsb):
        cid = lax.axis_index("c")
        sid = lax.axis_index("s")
        wid = sid * _NC + cid
        ebase = wid * per_w
        pltpu.sync_copy(dst_hbm.at[pl.ds(ebase, per_w)], idx_d)
        pltpu.sync_copy(src_hbm.at[pl.ds(ebase, per_w)], idx_s)

        def fire(ch, r1, r2, gs):
            off = ch * CH
            c1 = pltpu.async_copy(a_hbm.at[idx_d.at[pl.ds(off, CH)]], r1, gs)
            c2 = pltpu.async_copy(a_hbm.at[idx_s.at[pl.ds(off, CH)]], r2, gs)
            return c1, c2

        def sub(r1, r2, w):
            def row(r, carry):
                for j in range(H // 16):
                    s = pl.ds(j * 16, 16)
                    w[r, s] = r1[r, s] - r2[r, s]
                return carry

            lax.fori_loop(0, CH, row, 0)

        def wwait(w, ws):
            pltpu.make_async_copy(w, d_hbm.at[pl.ds(0, CH)], ws).wait()

        def proc(k_, ch, descs, r1, r2, w, ws):
            descs[0].wait()
            descs[1].wait()

            @pl.when(k_ > 0)
            def _():
                wwait(w, ws)

            sub(r1, r2, w)
            pltpu.async_copy(w, d_hbm.at[pl.ds(ebase + ch * CH, CH)], ws)

        def body(t, carry):
            ca = 2 * t
            cb = 2 * t + 1
            da = fire(ca, r1a, r2a, gsa)
            db = fire(cb, r1b, r2b, gsb)
            proc(t, ca, da, r1a, r2a, wa, wsa)
            proc(t, cb, db, r1b, r2b, wb, wsb)
            return carry

        lax.fori_loop(0, npairs, body, 0)
        dl = fire(nch - 1, r1a, r2a, gsa)
        proc(jnp.int32(1), nch - 1, dl, r1a, r2a, wa, wsa)
        wwait(wa, wsa)
        wwait(wb, wsb)

    return k(a, dst, src)


def _sc_scatter(e2, dst3):
    CH = 80
    nch = dst3.shape[1]
    npairs = nch // 2
    per_w = nch * CH
    ZR = 16
    rsmall, rbig = 624, 640

    @functools.partial(
        pl.kernel,
        out_type=SDS((2 * N1, H), F32),
        mesh=plsc.VectorSubcoreMesh(**_MESH),
        scratch_types=[pltpu.VMEM((2, 1, CH), I32),
                       pltpu.VMEM((CH, H), F32), pltpu.VMEM((CH, H), F32),
                       pltpu.VMEM((ZR, H), F32),
                       pltpu.VMEM_SHARED((N1, H), F32),
                       pltpu.SemaphoreType.DMA, pltpu.SemaphoreType.DMA],
    )
    def k(e2_hbm, dst_hbm, out_hbm, ibuf, ra, rb, zbuf, acc, sa, sb):
        cid = lax.axis_index("c")
        sid = lax.axis_index("s")
        wid = cid * _NS + sid
        nbase = cid * N1

        def zrow(r, carry):
            for j in range(H // 16):
                zbuf[r, pl.ds(j * 16, 16)] = jnp.zeros((16,), F32)
            return carry

        lax.fori_loop(0, ZR, zrow, 0)
        rbase = sid * rsmall
        last = _NS - 1

        def zcopy(t, carry):
            pltpu.sync_copy(zbuf, acc.at[pl.ds(rbase + t * ZR, ZR)])
            return carry

        lax.fori_loop(0, rsmall // ZR, zcopy, 0)

        @pl.when(sid == last)
        def _():
            pltpu.sync_copy(zbuf, acc.at[pl.ds(last * rsmall + rsmall, ZR)])

        plsc.subcore_barrier()

        def stage_idx(ch, slot):
            pltpu.sync_copy(dst_hbm.at[wid].at[ch], ibuf.at[slot])
            for j in range(CH // 16):
                s = pl.ds(j * 16, 16)
                ibuf[slot, 0, s] = ibuf[slot, 0, s] - nbase

        def fire(ch, rbuf, sem):
            off = wid * per_w + ch * CH
            return pltpu.async_copy(e2_hbm.at[pl.ds(off, CH)], rbuf, sem)

        def body(t, carry):
            ca = 2 * t
            cb = 2 * t + 1
            da = fire(ca, ra, sa)
            db = fire(cb, rb, sb)
            stage_idx(ca, 0)
            stage_idx(cb, 1)
            da.wait()
            pltpu.sync_copy(ra, acc.at[ibuf.at[0, 0]], add=True)
            db.wait()
            pltpu.sync_copy(rb, acc.at[ibuf.at[1, 0]], add=True)
            return carry

        lax.fori_loop(0, npairs, body, 0)
        dl = fire(nch - 1, ra, sa)
        stage_idx(nch - 1, 0)
        dl.wait()
        pltpu.sync_copy(ra, acc.at[ibuf.at[0, 0]], add=True)
        plsc.subcore_barrier()

        pltpu.sync_copy(acc.at[pl.ds(rbase, rsmall)],
                        out_hbm.at[pl.ds(nbase + rbase, rsmall)])

        @pl.when(sid == last)
        def _():
            pltpu.sync_copy(acc.at[pl.ds(last * rsmall + rsmall, ZR)],
                            out_hbm.at[pl.ds(nbase + last * rsmall + rsmall, ZR)])

    return k(e2, dst3)


def _meta(pe, px, pu, x_parts, e_parts, u_parts, gidx, attention=False):
    dst, dst3, src, src_a, src_b, cum, b_a, b_b = gidx
    npart = len(x_parts)
    K = npart * H

    (we1, be1), el2, el3 = _full_w(pe)
    (wx1, bx1), xl2, xl3 = _full_w(px)

    w1x = we1[0:K]
    w1e = we1[K:2 * K]
    w1u = we1[2 * K:3 * K]
    a = _a_proj(x_parts, w1x)
    d = _sc_gather_diff(a, dst, src)

    eouts = _edge(d, e_parts, src_a, src_b, cum, u_parts,
                  w1e, w1u, be1, el2, el3, with_ew=attention)
    if attention:
        e2, aggg, aggew = eouts
    else:
        e2, aggg = eouts
        aggew = None

    aggn = _sc_scatter(e2, dst3)

    w1xa = wx1[0:K + H]
    w1xu = wx1[K + H:K + H + npart * H]
    nouts = _node(x_parts, [aggn], b_a, b_b, u_parts,
                  w1xa, w1xu, bx1, xl2, xl3, with_xw=attention)
    if attention:
        x2, aggx, aggxw = nouts
    else:
        x2, aggx = nouts
        aggxw = None

    u2 = _useg(aggx, aggg, u_parts, _full_w(pu))
    return x2, e2, u2, (aggxw, aggew)


def kernel(x1, edge_index1, e1, u1, batch1, x2, edge_index2, e2, u2, batch2, params):
    x = jnp.concatenate([x1, x2], axis=0)
    e = jnp.concatenate([e1, e2], axis=0)
    u = jnp.concatenate([u1, u2], axis=0)
    batch = jnp.concatenate(
        [batch1.astype(I32), batch2.astype(I32) + 8], axis=0)
    src = jnp.concatenate(
        [edge_index1[0].astype(I32), edge_index2[0].astype(I32) + N1])
    dst = jnp.concatenate(
        [edge_index1[1].astype(I32), edge_index2[1].astype(I32) + N1])

    E = src.shape[0]
    N = x.shape[0]

    src_a = src.reshape(E // BLK, BLK, 1)
    src_b = src.reshape(E // BLK, 1, BLK)
    b_a = batch.reshape(N // BLK, BLK, 1)
    b_b = batch.reshape(N // BLK, 1, BLK)
    dst3 = dst.reshape(_NW, E // (_NW * 80), 1, 80)
    cum = _graph_starts(b_a, b_b)
    gidx = (dst, dst3, src, src_a, src_b, cum, b_a, b_b)

    p = params
    x_h = _enc(x, _full_w(p['enc_x']))
    e_h = _enc(e, _full_w(p['enc_e']))
    u_h = _enc(u, _full_w(p['enc_u']))

    for _ in range(3):
        x_h, e_h, u_h, _unused = _meta(
            p['rec_e'], p['rec_x'], p['rec_u'],
            [x, x_h], [e, e_h], [u, u_h], gidx, attention=False)

    _xa, _ea, u_a, (aggxw, aggew) = _meta(
        p['att_e'], p['att_x'], p['att_u'],
        [x_h], [e_h], [u_h], gidx, attention=True)

    return _final(aggxw, aggew, u_a, u_h,
                  _full_w(p['agg_u']), _full_w(p['final']))


# speedup vs baseline: 5.3343x; 1.0446x over previous
import functools

import jax
import jax.numpy as jnp
from jax import lax
from jax.experimental import pallas as pl
from jax.experimental.pallas import tpu as pltpu
from jax.experimental.pallas import tpu_sc as plsc

F32 = jnp.float32
I32 = jnp.int32
SDS = jax.ShapeDtypeStruct
PREC = lax.Precision.DEFAULT

N1 = 10000
E1 = 160000
NG = 16
H = 128

BLK = 10000


def _dot(a, b):
    return lax.dot_general(a, b, (((a.ndim - 1,), (0,)), ((), ())),
                           precision=PREC, preferred_element_type=F32)


def _mlp23(h, l2, l3):
    (w2, b2), (w3, b3) = l2, l3
    h = jnp.maximum(_dot(h, w2[...]) + b2[...], 0.0)
    return _dot(h, w3[...]) + b3[...]


def _wspec(arr):
    nd = arr.ndim
    return pl.BlockSpec(arr.shape, lambda i: (0,) * nd)


def _full_w(params):
    return [(w, b.reshape(1, -1)) for (w, b) in params]


def _enc(x, layers):
    R = x.shape[0]
    blk = BLK if R % BLK == 0 else R
    (w1, b1), l2, l3 = layers

    def body(x_ref, w1r, b1r, w2r, b2r, w3r, b3r, o_ref):
        h = jnp.maximum(_dot(x_ref[...], w1r[...]) + b1r[...], 0.0)
        o_ref[...] = _mlp23(h, (w2r, b2r), (w3r, b3r))

    ins = [x, w1, b1, l2[0], l2[1], l3[0], l3[1]]
    specs = [pl.BlockSpec((blk, x.shape[1]), lambda i: (i, 0))] + [_wspec(a) for a in ins[1:]]
    return pl.pallas_call(
        body, grid=(R // blk,), in_specs=specs,
        out_specs=pl.BlockSpec((blk, H), lambda i: (i, 0)),
        out_shape=SDS((R, H), F32))(*ins)


def _a_proj(x_parts, w1x):
    R = x_parts[0].shape[0]
    np_ = len(x_parts)

    def body(*refs):
        xr = refs[:np_]
        wr = refs[np_]
        o_ref = refs[np_ + 1]
        xc = jnp.concatenate([r[...] for r in xr], axis=1)
        o_ref[...] = _dot(xc, wr[...])

    ins = list(x_parts) + [w1x]
    specs = [pl.BlockSpec((BLK, H), lambda i: (i, 0)) for _ in x_parts] + [_wspec(w1x)]
    return pl.pallas_call(
        body, grid=(R // BLK,), in_specs=specs,
        out_specs=pl.BlockSpec((BLK, H), lambda i: (i, 0)),
        out_shape=SDS((R, H), F32))(*ins)


def _graph_starts(b_a, b_b):
    nblk = b_a.shape[0]

    def body(bar, bbr, row_ref, col_ref):
        i = pl.program_id(0)
        ids_a = bar[...][0]
        ids_b = bbr[...][0]
        lt = (ids_a < lax.broadcasted_iota(I32, (BLK, NG), 1)).astype(F32)
        ltT = (lax.broadcasted_iota(I32, (NG, BLK), 0) > ids_b).astype(F32)

        @pl.when(i == 0)
        def _():
            row_ref[...] = jnp.zeros((1, NG), F32)
            col_ref[...] = jnp.zeros((NG, 1), F32)

        row_ref[...] += _dot(jnp.ones((1, BLK), F32), lt)
        col_ref[...] += _dot(ltT, jnp.ones((BLK, 1), F32))

    return pl.pallas_call(
        body, grid=(nblk,),
        in_specs=[pl.BlockSpec((1, BLK, 1), lambda i: (i, 0, 0)),
                  pl.BlockSpec((1, 1, BLK), lambda i: (i, 0, 0))],
        out_specs=(pl.BlockSpec((1, NG), lambda i: (0, 0)),
                   pl.BlockSpec((NG, 1), lambda i: (0, 0))),
        out_shape=(SDS((1, NG), F32), SDS((NG, 1), F32)))(b_a, b_b)


def _seg_onehots(ids_a, ids_b, cum_row, cum_col):
    fa = ids_a.astype(F32)
    fb = ids_b.astype(F32)
    ge = (fa >= cum_row).astype(F32)
    geT = (fb >= cum_col).astype(F32)
    oh = ge - jnp.concatenate([ge[:, 1:], jnp.zeros((ge.shape[0], 1), F32)], 1)
    ohT = geT - jnp.concatenate([geT[1:, :], jnp.zeros((1, geT.shape[1]), F32)], 0)
    return oh, ohT


def _edge(d, e_parts, src_a, src_b, cum, u_parts, w1e, w1u, b1, l2, l3,
          with_ew=False):
    E = d.shape[0]
    cum_row, cum_col = cum
    ne, nu = len(e_parts), len(u_parts)

    def body(*refs):
        it = iter(refs)
        dr = next(it)
        ers = [next(it) for _ in range(ne)]
        sar = next(it); sbr = next(it)
        crr = next(it); ccr = next(it)
        urs = [next(it) for _ in range(nu)]
        w1er = next(it); w1ur = next(it); b1r = next(it)
        w2r = next(it); b2r = next(it); w3r = next(it); b3r = next(it)
        e2_ref = next(it)
        aggg_ref = next(it)
        aggew_ref = next(it) if with_ew else None

        i = pl.program_id(0)
        ecat = jnp.concatenate([r[...] for r in ers], axis=1)
        ucat = jnp.concatenate([r[...] for r in urs], axis=1)
        gu = _dot(ucat, w1ur[...]) + b1r[...]
        oh, ohT = _seg_onehots(sar[...][0], sbr[...][0], crr[...], ccr[...])
        h = dr[...] + _dot(ecat, w1er[...]) + _dot(oh, gu)
        h = jnp.maximum(h, 0.0)
        e2 = _mlp23(h, (w2r, b2r), (w3r, b3r))
        e2_ref[...] = e2

        @pl.when(i == 0)
        def _():
            aggg_ref[...] = jnp.zeros((NG, H), F32)
            if with_ew:
                aggew_ref[...] = jnp.zeros((NG, H), F32)

        aggg_ref[...] += _dot(ohT, e2)
        if with_ew:
            aggew_ref[...] += _dot(ohT, e2 * ecat)

    ins = [d] + list(e_parts) + [src_a, src_b, cum_row, cum_col] + \
        list(u_parts) + [w1e, w1u, b1, l2[0], l2[1], l3[0], l3[1]]
    specs = [pl.BlockSpec((BLK, H), lambda i: (i, 0))]
    specs += [pl.BlockSpec((BLK, p.shape[1]), lambda i: (i, 0)) for p in e_parts]
    specs += [pl.BlockSpec((1, BLK, 1), lambda i: (i, 0, 0)),
              pl.BlockSpec((1, 1, BLK), lambda i: (i, 0, 0))]
    specs += [_wspec(a) for a in ins[3 + ne:]]

    out_shape = [SDS((E, H), F32), SDS((NG, H), F32)]
    out_specs = [pl.BlockSpec((BLK, H), lambda i: (i, 0)),
                 pl.BlockSpec((NG, H), lambda i: (0, 0))]
    if with_ew:
        out_shape.append(SDS((NG, H), F32))
        out_specs.append(pl.BlockSpec((NG, H), lambda i: (0, 0)))
    return pl.pallas_call(
        body, grid=(E // BLK,), in_specs=specs, out_specs=tuple(out_specs),
        out_shape=tuple(out_shape))(*ins)


def _node(x_parts, aggn_parts, b_a, b_b, u_parts, w1xa, w1u, b1, l2, l3,
          with_xw=False):
    R = x_parts[0].shape[0]
    nx, na, nu = len(x_parts), len(aggn_parts), len(u_parts)

    def body(*refs):
        it = iter(refs)
        xrs = [next(it) for _ in range(nx)]
        ars = [next(it) for _ in range(na)]
        bar = next(it); bbr = next(it)
        urs = [next(it) for _ in range(nu)]
        w1r = next(it); w1ur = next(it); b1r = next(it)
        w2r = next(it); b2r = next(it); w3r = next(it); b3r = next(it)
        x2_ref = next(it)
        aggx_ref = next(it)
        aggxw_ref = next(it) if with_xw else None

        i = pl.program_id(0)
        xpcat = jnp.concatenate([r[...] for r in xrs], axis=1)
        xc = jnp.concatenate([xpcat] + [r[...] for r in ars], axis=1)
        ucat = jnp.concatenate([r[...] for r in urs], axis=1)
        gu = _dot(ucat, w1ur[...]) + b1r[...]
        ids_a = bar[...][0]
        ids_b = bbr[...][0]
        oh = (ids_a == lax.broadcasted_iota(I32, (BLK, NG), 1)).astype(F32)
        ohT = (ids_b == lax.broadcasted_iota(I32, (NG, BLK), 0)).astype(F32)
        h = jnp.maximum(_dot(xc, w1r[...]) + _dot(oh, gu), 0.0)
        x2 = _mlp23(h, (w2r, b2r), (w3r, b3r))
        x2_ref[...] = x2

        @pl.when(i == 0)
        def _():
            aggx_ref[...] = jnp.zeros((NG, H), F32)
            if with_xw:
                aggxw_ref[...] = jnp.zeros((NG, H), F32)

        aggx_ref[...] += _dot(ohT, x2)
        if with_xw:
            aggxw_ref[...] += _dot(ohT, x2 * xpcat)

    ins = list(x_parts) + list(aggn_parts) + [b_a, b_b] + list(u_parts) + \
        [w1xa, w1u, b1, l2[0], l2[1], l3[0], l3[1]]
    specs = [pl.BlockSpec((BLK, p.shape[1]), lambda i: (i, 0))
             for p in list(x_parts) + list(aggn_parts)]
    specs += [pl.BlockSpec((1, BLK, 1), lambda i: (i, 0, 0)),
              pl.BlockSpec((1, 1, BLK), lambda i: (i, 0, 0))]
    specs += [_wspec(a) for a in ins[nx + na + 2:]]

    out_shape = [SDS((R, H), F32), SDS((NG, H), F32)]
    out_specs = [pl.BlockSpec((BLK, H), lambda i: (i, 0)),
                 pl.BlockSpec((NG, H), lambda i: (0, 0))]
    if with_xw:
        out_shape.append(SDS((NG, H), F32))
        out_specs.append(pl.BlockSpec((NG, H), lambda i: (0, 0)))
    return pl.pallas_call(
        body, grid=(R // BLK,), in_specs=specs, out_specs=tuple(out_specs),
        out_shape=tuple(out_shape))(*ins)


def _useg(aggx, aggg, u_parts, layers):
    nu = len(u_parts)
    (w1, b1), l2, l3 = layers

    def body(*refs):
        it = iter(refs)
        ax = next(it); ag = next(it)
        urs = [next(it) for _ in range(nu)]
        w1r = next(it); b1r = next(it)
        w2r = next(it); b2r = next(it); w3r = next(it); b3r = next(it)
        o_ref = next(it)
        xin = jnp.concatenate([ax[...], ag[...]] + [r[...] for r in urs], axis=1)
        h = jnp.maximum(_dot(xin, w1r[...]) + b1r[...], 0.0)
        o_ref[...] = _mlp23(h, (w2r, b2r), (w3r, b3r))

    ins = [aggx, aggg] + list(u_parts) + [w1, b1, l2[0], l2[1], l3[0], l3[1]]
    specs = [_wspec(a) for a in ins]
    return pl.pallas_call(
        body, grid=(1,), in_specs=specs,
        out_specs=pl.BlockSpec((NG, H), lambda i: (0, 0)),
        out_shape=SDS((NG, H), F32))(*ins)


def _final(aggxw, aggew, u_a, u_h, lay_agg, lay_fin):
    def body(axw, aew, uar, uhr,
             aw1, ab1, aw2, ab2, aw3, ab3,
             fw1, fb1, fw2, fb2, fw3, fb3, o_ref):
        uw = uar[...] * uhr[...]
        gin = jnp.concatenate([axw[...], aew[...], uw], axis=1)
        h = jnp.maximum(_dot(gin, aw1[...]) + ab1[...], 0.0)
        g = _mlp23(h, (aw2, ab2), (aw3, ab3))
        d = g[0:8, :] - g[8:16, :]
        h2 = jnp.maximum(_dot(d, fw1[...]) + fb1[...], 0.0)
        o_ref[...] = _mlp23(h2, (fw2, fb2), (fw3, fb3))

    (aw1, ab1), (aw2, ab2), (aw3, ab3) = lay_agg
    (fw1, fb1), (fw2, fb2), (fw3, fb3) = lay_fin
    ins = [aggxw, aggew, u_a, u_h,
           aw1, ab1, aw2, ab2, aw3, ab3, fw1, fb1, fw2, fb2, fw3, fb3]
    specs = [_wspec(a) for a in ins]
    return pl.pallas_call(
        body, grid=(1,), in_specs=specs,
        out_specs=pl.BlockSpec((8, 64), lambda i: (0, 0)),
        out_shape=SDS((8, 64), F32))(*ins)


_MESH = dict(core_axis_name="c", subcore_axis_name="s")
_NC, _NS = 2, 16
_NW = _NC * _NS


def _sc_gather_diff(a, dst, src):
    E = dst.shape[0]
    CH = 80
    per_w = E // _NW
    nch = per_w // CH
    npairs = nch // 2

    @functools.partial(
        pl.kernel,
        out_type=SDS((E, H), F32),
        mesh=plsc.VectorSubcoreMesh(**_MESH),
        scratch_types=[pltpu.VMEM((per_w,), I32), pltpu.VMEM((per_w,), I32),
                       pltpu.VMEM((CH, H), F32), pltpu.VMEM((CH, H), F32),
                       pltpu.VMEM((CH, H), F32),
                       pltpu.VMEM((CH, H), F32), pltpu.VMEM((CH, H), F32),
                       pltpu.VMEM((CH, H), F32),
                       pltpu.SemaphoreType.DMA, pltpu.SemaphoreType.DMA,
                       pltpu.SemaphoreType.DMA, pltpu.SemaphoreType.DMA],
    )
    def k(a_hbm, dst_hbm, src_hbm, d_hbm,
          idx_d, idx_s, r1a, r2a, wa, r1b, r2b, wb, gsa, gsb, wsa, wsb):
        cid = lax.axis_index("c")
        sid = lax.axis_index("s")
        wid = sid * _NC + cid
        ebase = wid * per_w
        pltpu.sync_copy(dst_hbm.at[pl.ds(ebase, per_w)], idx_d)
        pltpu.sync_copy(src_hbm.at[pl.ds(ebase, per_w)], idx_s)

        def fire(ch, r1, r2, gs):
            off = ch * CH
            c1 = pltpu.async_copy(a_hbm.at[idx_d.at[pl.ds(off, CH)]], r1, gs)
            c2 = pltpu.async_copy(a_hbm.at[idx_s.at[pl.ds(off, CH)]], r2, gs)
            return c1, c2

        def sub(r1, r2, w):
            def row(r, carry):
                for j in range(H // 16):
                    s = pl.ds(j * 16, 16)
                    w[r, s] = r1[r, s] - r2[r, s]
                return carry

            lax.fori_loop(0, CH, row, 0)

        def wwait(w, ws):
            pltpu.make_async_copy(w, d_hbm.at[pl.ds(0, CH)], ws).wait()

        def proc(k_, ch, descs, r1, r2, w, ws):
            descs[0].wait()
            descs[1].wait()

            @pl.when(k_ > 0)
            def _():
                wwait(w, ws)

            sub(r1, r2, w)
            pltpu.async_copy(w, d_hbm.at[pl.ds(ebase + ch * CH, CH)], ws)

        def body(t, carry):
            ca = 2 * t
            cb = 2 * t + 1
            da = fire(ca, r1a, r2a, gsa)
            db = fire(cb, r1b, r2b, gsb)
            proc(t, ca, da, r1a, r2a, wa, wsa)
            proc(t, cb, db, r1b, r2b, wb, wsb)
            return carry

        lax.fori_loop(0, npairs, body, 0)
        dl = fire(nch - 1, r1a, r2a, gsa)
        proc(jnp.int32(1), nch - 1, dl, r1a, r2a, wa, wsa)
        wwait(wa, wsa)
        wwait(wb, wsb)

    return k(a, dst, src)


def _sc_scatter(e2, dst3):
    CH = 80
    nch = dst3.shape[1]
    npairs = nch // 2
    per_w = nch * CH
    ZR = 16
    rsmall, rbig = 624, 640

    @functools.partial(
        pl.kernel,
        out_type=SDS((2 * N1, H), F32),
        mesh=plsc.VectorSubcoreMesh(**_MESH),
        scratch_types=[pltpu.VMEM((2, 1, CH), I32),
                       pltpu.VMEM((CH, H), F32), pltpu.VMEM((CH, H), F32),
                       pltpu.VMEM((ZR, H), F32),
                       pltpu.VMEM_SHARED((N1, H), F32),
                       pltpu.SemaphoreType.DMA, pltpu.SemaphoreType.DMA],
    )
    def k(e2_hbm, dst_hbm, out_hbm, ibuf, ra, rb, zbuf, acc, sa, sb):
        cid = lax.axis_index("c")
        sid = lax.axis_index("s")
        wid = cid * _NS + sid
        nbase = cid * N1

        def zrow(r, carry):
            for j in range(H // 16):
                zbuf[r, pl.ds(j * 16, 16)] = jnp.zeros((16,), F32)
            return carry

        lax.fori_loop(0, ZR, zrow, 0)
        rbase = sid * rsmall
        last = _NS - 1

        def zcopy(t, carry):
            pltpu.sync_copy(zbuf, acc.at[pl.ds(rbase + t * ZR, ZR)])
            return carry

        lax.fori_loop(0, rsmall // ZR, zcopy, 0)

        @pl.when(sid == last)
        def _():
            pltpu.sync_copy(zbuf, acc.at[pl.ds(last * rsmall + rsmall, ZR)])

        plsc.subcore_barrier()

        def stage_idx(ch, slot):
            pltpu.sync_copy(dst_hbm.at[wid].at[ch], ibuf.at[slot])
            for j in range(CH // 16):
                s = pl.ds(j * 16, 16)
                ibuf[slot, 0, s] = ibuf[slot, 0, s] - nbase

        def fire(ch, rbuf, sem):
            off = wid * per_w + ch * CH
            return pltpu.async_copy(e2_hbm.at[pl.ds(off, CH)], rbuf, sem)

        def body(t, carry):
            ca = 2 * t
            cb = 2 * t + 1
            da = fire(ca, ra, sa)
            db = fire(cb, rb, sb)
            stage_idx(ca, 0)
            stage_idx(cb, 1)
            da.wait()
            pltpu.sync_copy(ra, acc.at[ibuf.at[0, 0]], add=True)
            db.wait()
            pltpu.sync_copy(rb, acc.at[ibuf.at[1, 0]], add=True)
            return carry

        lax.fori_loop(0, npairs, body, 0)
        dl = fire(nch - 1, ra, sa)
        stage_idx(nch - 1, 0)
        dl.wait()
        pltpu.sync_copy(ra, acc.at[ibuf.at[0, 0]], add=True)
        plsc.subcore_barrier()

        pltpu.sync_copy(acc.at[pl.ds(rbase, rsmall)],
                        out_hbm.at[pl.ds(nbase + rbase, rsmall)])

        @pl.when(sid == last)
        def _():
            pltpu.sync_copy(acc.at[pl.ds(last * rsmall + rsmall, ZR)],
                            out_hbm.at[pl.ds(nbase + last * rsmall + rsmall, ZR)])

    return k(e2, dst3)


def _meta(pe, px, pu, x_parts, e_parts, u_parts, gidx, attention=False):
    dst, dst3, src, src_a, src_b, cum, b_a, b_b = gidx
    npart = len(x_parts)
    K = npart * H

    (we1, be1), el2, el3 = _full_w(pe)
    (wx1, bx1), xl2, xl3 = _full_w(px)

    w1x = we1[0:K]
    w1e = we1[K:2 * K]
    w1u = we1[2 * K:3 * K]
    a = _a_proj(x_parts, w1x)
    d = _sc_gather_diff(a, dst, src)

    eouts = _edge(d, e_parts, src_a, src_b, cum, u_parts,
                  w1e, w1u, be1, el2, el3, with_ew=attention)
    if attention:
        e2, aggg, aggew = eouts
    else:
        e2, aggg = eouts
        aggew = None

    aggn = _sc_scatter(e2, dst3)

    w1xa = wx1[0:K + H]
    w1xu = wx1[K + H:K + H + npart * H]
    nouts = _node(x_parts, [aggn], b_a, b_b, u_parts,
                  w1xa, w1xu, bx1, xl2, xl3, with_xw=attention)
    if attention:
        x2, aggx, aggxw = nouts
    else:
        x2, aggx = nouts
        aggxw = None

    u2 = _useg(aggx, aggg, u_parts, _full_w(pu))
    return x2, e2, u2, (aggxw, aggew)


def kernel(x1, edge_index1, e1, u1, batch1, x2, edge_index2, e2, u2, batch2, params):
    x = jnp.concatenate([x1, x2], axis=0)
    e = jnp.concatenate([e1, e2], axis=0)
    u = jnp.concatenate([u1, u2], axis=0)
    batch = jnp.concatenate(
        [batch1.astype(I32), batch2.astype(I32) + 8], axis=0)
    src = jnp.concatenate(
        [edge_index1[0].astype(I32), edge_index2[0].astype(I32) + N1])
    dst = jnp.concatenate(
        [edge_index1[1].astype(I32), edge_index2[1].astype(I32) + N1])

    E = src.shape[0]
    N = x.shape[0]

    src_a = src.reshape(E // BLK, BLK, 1)
    src_b = src.reshape(E // BLK, 1, BLK)
    b_a = batch.reshape(N // BLK, BLK, 1)
    b_b = batch.reshape(N // BLK, 1, BLK)
    dst3 = dst.reshape(_NW, E // (_NW * 80), 1, 80)
    cum = _graph_starts(b_a, b_b)
    gidx = (dst, dst3, src, src_a, src_b, cum, b_a, b_b)

    p = params
    x_h = _enc(x, _full_w(p['enc_x']))
    e_h = _enc(e, _full_w(p['enc_e']))
    u_h = _enc(u, _full_w(p['enc_u']))

    for _ in range(3):
        x_h, e_h, u_h, _unused = _meta(
            p['rec_e'], p['rec_x'], p['rec_u'],
            [x, x_h], [e, e_h], [u, u_h], gidx, attention=False)

    _xa, _ea, u_a, (aggxw, aggew) = _meta(
        p['att_e'], p['att_x'], p['att_u'],
        [x_h], [e_h], [u_h], gidx, attention=True)

    return _final(aggxw, aggew, u_a, u_h,
                  _full_w(p['agg_u']), _full_w(p['final']))
